# baseline (speedup 1.0000x reference)
import functools

import jax
import jax.numpy as jnp
from jax import lax
from jax.experimental import pallas as pl
from jax.experimental.pallas import tpu as pltpu
from jax.experimental.pallas import tpu_sc as plsc

N = 10000
B = 4
NB = N // B
E = 160000
ETOT = E + N
EB = ETOT // B
BLK = 1024
G = (ETOT + BLK - 1) // BLK
EPAD = G * BLK
NPAD = 10240
NW = 32
EW = EPAD // NW
NEG = -1.0e30


def _lrelu(v, s):
    return jnp.where(v >= 0, v, s * v)


def _k1_body(x_ref, wn_ref, bn_ref, wl_ref, bl_ref, wr_ref, br_ref,
             xl_ref, xlb_ref, xrb_ref):
    h = _lrelu(x_ref[...] * wn_ref[...] + bn_ref[...], 0.01)
    xl = jnp.dot(h, wl_ref[...], preferred_element_type=jnp.float32) + bl_ref[...]
    xr = jnp.dot(h, wr_ref[...], preferred_element_type=jnp.float32) + br_ref[...]
    xl_ref[...] = xl
    xlb_ref[...] = xl.astype(jnp.bfloat16)
    xrb_ref[...] = xr.astype(jnp.bfloat16)


def _k1(x2, W_node, b_node, W_l, b_l, W_r, b_r):
    n_blk = NPAD // BLK
    full = lambda shape: pl.BlockSpec(shape, lambda g: (0, 0))
    return pl.pallas_call(
        _k1_body,
        grid=(n_blk,),
        in_specs=[
            pl.BlockSpec((BLK, 1), lambda g: (g, 0)),
            full((1, 512)), full((1, 512)),
            full((512, 1024)), full((1, 1024)),
            full((512, 1024)), full((1, 1024)),
        ],
        out_specs=[
            pl.BlockSpec((BLK, 1024), lambda g: (g, 0)),
            pl.BlockSpec((BLK, 1024), lambda g: (g, 0)),
            pl.BlockSpec((BLK, 1024), lambda g: (g, 0)),
        ],
        out_shape=[
            jax.ShapeDtypeStruct((NPAD, 1024), jnp.float32),
            jax.ShapeDtypeStruct((NPAD, 1024), jnp.bfloat16),
            jax.ShapeDtypeStruct((NPAD, 1024), jnp.bfloat16),
        ],
    )(x2, W_node, b_node, W_l, b_l, W_r, b_r)


def _g1_body(xl_hbm, xr_hbm, src_hbm, dst_hbm, gl_hbm, gr_hbm,
             src_v, dst_v, bufa, bufb, sema, semb):
    wid = lax.axis_index("s") * 2 + lax.axis_index("c")
    base = pl.multiple_of(wid * EW, 8)
    pltpu.sync_copy(src_hbm.at[pl.ds(base, EW)], src_v)
    pltpu.sync_copy(dst_hbm.at[pl.ds(base, EW)], dst_v)

    C = 64
    NFULL = EW // C

    def chunk(rows, off):
        da = pltpu.async_copy(xl_hbm.at[src_v.at[pl.ds(off, rows)]],
                              bufa.at[pl.ds(0, rows)], sema)
        db = pltpu.async_copy(xr_hbm.at[dst_v.at[pl.ds(off, rows)]],
                              bufb.at[pl.ds(0, rows)], semb)
        da.wait()
        db.wait()
        pltpu.sync_copy(bufa.at[pl.ds(0, rows)],
                        gl_hbm.at[pl.ds(base + off, rows)])
        pltpu.sync_copy(bufb.at[pl.ds(0, rows)],
                        gr_hbm.at[pl.ds(base + off, rows)])

    def body(k, _):
        chunk(C, k * C)
        return _

    lax.fori_loop(0, NFULL, body, None)
    chunk(EW - NFULL * C, NFULL * C)


def _g1(xl_i, xr_i, src, dst):
    mesh = plsc.VectorSubcoreMesh(core_axis_name="c", subcore_axis_name="s")
    f = pl.kernel(
        _g1_body,
        out_type=(
            jax.ShapeDtypeStruct((EPAD, 512), jnp.int32),
            jax.ShapeDtypeStruct((EPAD, 512), jnp.int32),
        ),
        mesh=mesh,
        compiler_params=pltpu.CompilerParams(needs_layout_passes=False),
        scratch_types=[
            pltpu.VMEM((EW,), jnp.int32),
            pltpu.VMEM((EW,), jnp.int32),
            pltpu.VMEM((64, 512), jnp.int32),
            pltpu.VMEM((64, 512), jnp.int32),
            pltpu.SemaphoreType.DMA,
            pltpu.SemaphoreType.DMA,
        ],
    )
    return f(xl_i, xr_i, src, dst)


def _k2_body(ea_ref, gl_ref, gr_ref, dst_ref, ec_ref, wep_ref, bep_ref,
             we_ref, att_ref, lg_ref, ehd_ref, ehp_ref, mx_ref):
    g = pl.program_id(0)
    eh = _lrelu(jnp.dot(ea_ref[...], wep_ref[...],
                        preferred_element_type=jnp.float32) + bep_ref[...], 0.01)
    z = jnp.dot(eh.astype(jnp.bfloat16), we_ref[...],
                preferred_element_type=jnp.float32)
    v = (z + gl_ref[...].astype(jnp.float32)
         + gr_ref[...].astype(jnp.float32))
    p = _lrelu(v, 0.2)
    lg = jnp.sum(p * att_ref[...], axis=1, keepdims=True)

    col = jax.lax.broadcasted_iota(jnp.int32, (BLK, 1), 0) + g * BLK
    lgv = jnp.where(col < ETOT, lg, NEG)
    lg_ref[0] = lgv

    bm = jnp.max(lgv)
    prev = jnp.where(g == 0, NEG, mx_ref[...])
    mx_ref[...] = jnp.maximum(prev, jnp.full((1, 128), bm))

    row = jax.lax.broadcasted_iota(jnp.int32, (1, BLK), 1) + g * BLK
    valid2 = row < ETOT
    lanes = jax.lax.broadcasted_iota(jnp.int32, (B, BLK), 0)
    dstc = dst_ref[0] // NB
    oh_d = jnp.where((dstc == lanes) & valid2, 1.0, 0.0)
    oh_p = jnp.where((ec_ref[0] == lanes) & valid2, 1.0, 0.0)
    eh_aug = jnp.concatenate([eh, jnp.ones((BLK, 1), jnp.float32)], axis=1)

    d_contrib = jnp.dot(oh_d, eh_aug, preferred_element_type=jnp.float32)
    p_contrib = jnp.dot(oh_p, eh_aug, preferred_element_type=jnp.float32)
    ehd_ref[...] = jnp.where(g == 0, 0.0, ehd_ref[...]) + d_contrib
    ehp_ref[...] = jnp.where(g == 0, 0.0, ehp_ref[...]) + p_contrib


def _k2(ea_pad, gl, gr, dst3, ec3, W_ep, b_ep, W_e, att):
    full = lambda shape: pl.BlockSpec(shape, lambda g: (0, 0))
    return pl.pallas_call(
        _k2_body,
        grid=(G,),
        in_specs=[
            pl.BlockSpec((BLK, 5), lambda g: (g, 0)),
            pl.BlockSpec((BLK, 1024), lambda g: (g, 0)),
            pl.BlockSpec((BLK, 1024), lambda g: (g, 0)),

            pl.BlockSpec((1, 1, BLK), lambda g: (g, 0, 0)),
            pl.BlockSpec((1, 1, BLK), lambda g: (g, 0, 0)),
            full((5, 511)), full((1, 511)), full((511, 1024)), full((1, 1024)),
        ],
        out_specs=[
            pl.BlockSpec((1, BLK, 1), lambda g: (g, 0, 0)),
            full((B, 512)), full((B, 512)),
            pl.BlockSpec((1, 128), lambda g: (0, 0)),
        ],
        out_shape=[
            jax.ShapeDtypeStruct((G, BLK, 1), jnp.float32),
            jax.ShapeDtypeStruct((B, 512), jnp.float32),
            jax.ShapeDtypeStruct((B, 512), jnp.float32),
            jax.ShapeDtypeStruct((1, 128), jnp.float32),
        ],
        compiler_params=pltpu.CompilerParams(
            dimension_semantics=("arbitrary",)),
    )(ea_pad, gl, gr, dst3, ec3, W_ep, b_ep, W_e, att)


def _k3_body(lg_hbm, src_hbm, dst_hbm, mx_hbm,
             ex_hbm, den_hbm, deg_hbm, c_hbm,
             lgv, srcv, dstv, mxv, denv, degv, cv):
    wid = lax.axis_index("s") * 2 + lax.axis_index("c")
    base = pl.multiple_of(wid * EW, 8)
    pltpu.sync_copy(lg_hbm.at[pl.ds(base, EW)], lgv)
    pltpu.sync_copy(src_hbm.at[pl.ds(base, EW)], srcv)
    pltpu.sync_copy(dst_hbm.at[pl.ds(base, EW)], dstv)
    pltpu.sync_copy(mx_hbm, mxv)

    def zero(i, ref):
        ref[pl.ds(i * 16, 16)] = jnp.zeros((16,), jnp.float32)

    lax.fori_loop(0, NPAD // 16, lambda i, _: (zero(i, denv), _)[1], None)
    lax.fori_loop(0, NPAD // 16, lambda i, _: (zero(i, degv), _)[1], None)
    lax.fori_loop(0, 4 * NPAD // 16, lambda i, _: (zero(i, cv), _)[1], None)

    mx16 = mxv[...]
    lane = jnp.arange(16, dtype=jnp.int32)
    ones = jnp.ones((16,), jnp.float32)

    def body(i, _):
        sl = pl.ds(i * 16, 16)
        ex16 = jnp.exp(lgv[sl] - mx16)
        lgv[sl] = ex16
        d16 = dstv[sl]
        s16 = srcv[sl]
        valid = (base + i * 16 + lane) < ETOT
        plsc.addupdate_scatter(denv, [d16], ex16)
        plsc.addupdate_scatter(degv, [d16], ones, mask=valid)
        m16 = d16 // NB
        plsc.addupdate_scatter(cv, [m16 * NPAD + s16], ones, mask=valid)
        return _

    lax.fori_loop(0, EW // 16, body, None)

    pltpu.sync_copy(lgv, ex_hbm.at[pl.ds(base, EW)])
    pltpu.sync_copy(denv, den_hbm.at[wid])
    pltpu.sync_copy(degv, deg_hbm.at[wid])
    pltpu.sync_copy(cv, c_hbm.at[wid])


def _k3(lg, src, dst, mx16):
    mesh = plsc.VectorSubcoreMesh(core_axis_name="c", subcore_axis_name="s")
    f = pl.kernel(
        _k3_body,
        out_type=(
            jax.ShapeDtypeStruct((EPAD,), jnp.float32),
            jax.ShapeDtypeStruct((NW, NPAD), jnp.float32),
            jax.ShapeDtypeStruct((NW, NPAD), jnp.float32),
            jax.ShapeDtypeStruct((NW, 4 * NPAD), jnp.float32),
        ),
        mesh=mesh,
        compiler_params=pltpu.CompilerParams(needs_layout_passes=False),
        scratch_types=[
            pltpu.VMEM((EW,), jnp.float32),
            pltpu.VMEM((EW,), jnp.int32),
            pltpu.VMEM((EW,), jnp.int32),
            pltpu.VMEM((16,), jnp.float32),
            pltpu.VMEM((NPAD,), jnp.float32),
            pltpu.VMEM((NPAD,), jnp.float32),
            pltpu.VMEM((4 * NPAD,), jnp.float32),
        ],
    )
    return f(lg, src, dst, mx16)


def _k4_body(denp_ref, degp_ref, cp_ref, den_ref, deg_ref, c_ref):
    den_ref[...] = jnp.sum(denp_ref[...], axis=0, keepdims=True)
    deg_ref[...] = jnp.sum(degp_ref[...], axis=0, keepdims=True)
    c_ref[...] = jnp.sum(cp_ref[...], axis=0, keepdims=True)


def _k4(den_p, deg_p, c_p):
    return pl.pallas_call(
        _k4_body,
        out_shape=[
            jax.ShapeDtypeStruct((1, NPAD), jnp.float32),
            jax.ShapeDtypeStruct((1, NPAD), jnp.float32),
            jax.ShapeDtypeStruct((1, 4 * NPAD), jnp.float32),
        ],
    )(den_p, deg_p, c_p)


def _k5a_body(ex_hbm, src_hbm, dst_hbm, den_hbm, deg_hbm,
              al_hbm, a1_hbm, a3_hbm, asum_hbm,
              exv, srcv, dstv, denv, degv, a1v, a3v, asumv):
    wid = lax.axis_index("s") * 2 + lax.axis_index("c")
    base = pl.multiple_of(wid * EW, 8)
    pltpu.sync_copy(ex_hbm.at[pl.ds(base, EW)], exv)
    pltpu.sync_copy(src_hbm.at[pl.ds(base, EW)], srcv)
    pltpu.sync_copy(dst_hbm.at[pl.ds(base, EW)], dstv)
    pltpu.sync_copy(den_hbm, denv)
    pltpu.sync_copy(deg_hbm, degv)

    def zero(i, ref):
        ref[pl.ds(i * 16, 16)] = jnp.zeros((16,), jnp.float32)

    lax.fori_loop(0, 4 * NPAD // 16, lambda i, _: (zero(i, a1v), _)[1], None)
    lax.fori_loop(0, 4 * NPAD // 16, lambda i, _: (zero(i, a3v), _)[1], None)
    for b in range(8):
        asumv[pl.ds(b * 16, 16)] = jnp.zeros((16,), jnp.float32)

    lane = jnp.arange(16, dtype=jnp.int32)
    zf = jnp.zeros((16,), jnp.float32)

    def body(i, _):
        sl = pl.ds(i * 16, 16)
        d16 = dstv[sl]
        s16 = srcv[sl]
        den16 = plsc.load_gather(denv, [d16])
        a16 = exv[sl] / den16
        exv[sl] = a16
        deg16 = plsc.load_gather(degv, [d16])
        m16 = d16 // NB
        fl = m16 * NPAD + s16
        plsc.addupdate_scatter(a1v, [fl], a16)
        plsc.addupdate_scatter(a3v, [fl], deg16 * a16)
        p16 = (base + i * 16 + lane) // EB
        for b in range(B):
            plsc.addupdate(asumv.at[pl.ds(b * 16, 16)],
                           jnp.where(m16 == b, a16, zf))
            plsc.addupdate(asumv.at[pl.ds((4 + b) * 16, 16)],
                           jnp.where(p16 == b, a16, zf))
        return _

    lax.fori_loop(0, EW // 16, body, None)

    pltpu.sync_copy(exv, al_hbm.at[pl.ds(base, EW)])
    pltpu.sync_copy(a1v, a1_hbm.at[wid])
    pltpu.sync_copy(a3v, a3_hbm.at[wid])
    pltpu.sync_copy(asumv, asum_hbm.at[wid])


def _k5a(ex, src, dst, den1, deg1):
    mesh = plsc.VectorSubcoreMesh(core_axis_name="c", subcore_axis_name="s")
    f = pl.kernel(
        _k5a_body,
        out_type=(
            jax.ShapeDtypeStruct((EPAD,), jnp.float32),
            jax.ShapeDtypeStruct((NW, 4 * NPAD), jnp.float32),
            jax.ShapeDtypeStruct((NW, 4 * NPAD), jnp.float32),
            jax.ShapeDtypeStruct((NW, 128), jnp.float32),
        ),
        mesh=mesh,
        compiler_params=pltpu.CompilerParams(needs_layout_passes=False),
        scratch_types=[
            pltpu.VMEM((EW,), jnp.float32),
            pltpu.VMEM((EW,), jnp.int32),
            pltpu.VMEM((EW,), jnp.int32),
            pltpu.VMEM((NPAD,), jnp.float32),
            pltpu.VMEM((NPAD,), jnp.float32),
            pltpu.VMEM((4 * NPAD,), jnp.float32),
            pltpu.VMEM((4 * NPAD,), jnp.float32),
            pltpu.VMEM((128,), jnp.float32),
        ],
    )
    return f(ex, src, dst, den1, deg1)


def _k5b_body(al_hbm, src_hbm, dst_hbm, c_hbm, a2_hbm,
              alv, srcv, dstv, cv, a2v):
    wid = lax.axis_index("s") * 2 + lax.axis_index("c")
    base = pl.multiple_of(wid * EW, 8)
    pltpu.sync_copy(al_hbm.at[pl.ds(base, EW)], alv)
    pltpu.sync_copy(src_hbm.at[pl.ds(base, EW)], srcv)
    pltpu.sync_copy(dst_hbm.at[pl.ds(base, EW)], dstv)
    pltpu.sync_copy(c_hbm, cv)

    def zero(i, ref):
        ref[pl.ds(i * 16, 16)] = jnp.zeros((16,), jnp.float32)

    lax.fori_loop(0, 4 * NPAD // 16, lambda i, _: (zero(i, a2v), _)[1], None)

    def body(i, _):
        sl = pl.ds(i * 16, 16)
        a16 = alv[sl]
        d16 = dstv[sl]
        s16 = srcv[sl]
        for b in range(B):
            cb = plsc.load_gather(cv, [b * NPAD + d16])
            plsc.addupdate_scatter(a2v, [b * NPAD + s16], cb * a16)
        return _

    lax.fori_loop(0, EW // 16, body, None)
    pltpu.sync_copy(a2v, a2_hbm.at[wid])


def _k5b(al, src, dst, c1):
    mesh = plsc.VectorSubcoreMesh(core_axis_name="c", subcore_axis_name="s")
    f = pl.kernel(
        _k5b_body,
        out_type=jax.ShapeDtypeStruct((NW, 4 * NPAD), jnp.float32),
        mesh=mesh,
        compiler_params=pltpu.CompilerParams(needs_layout_passes=False),
        scratch_types=[
            pltpu.VMEM((EW,), jnp.float32),
            pltpu.VMEM((EW,), jnp.int32),
            pltpu.VMEM((EW,), jnp.int32),
            pltpu.VMEM((4 * NPAD,), jnp.float32),
            pltpu.VMEM((4 * NPAD,), jnp.float32),
        ],
    )
    return f(al, src, dst, c1)


def _k6_body(a1_ref, a2_ref, a3_ref, xl_ref, ehd_ref, ehp_ref, asum_ref,
             y2_ref, wgp_ref, bgp_ref, bgat_ref, wmsg_ref, bmsg_ref,
             wmsgi_ref, bmsgi_ref, wge_ref, bge_ref, wf1_ref, bf1_ref,
             wf2_ref, bf2_ref, ss_ref, out_ref):
    j = pl.program_id(0)
    a1 = jnp.sum(a1_ref[...], axis=0)
    a2 = jnp.sum(a2_ref[...], axis=0)
    a3 = jnp.sum(a3_ref[...], axis=0)
    astack = jnp.concatenate([a1, a2, a3], axis=0)
    contrib = jnp.dot(astack, xl_ref[...], preferred_element_type=jnp.float32)
    ss_ref[...] = jnp.where(j == 0, 0.0, ss_ref[...]) + contrib

    @pl.when(j == pl.num_programs(0) - 1)
    def _():
        ss = ss_ref[...]
        s1, s2, s3 = ss[0:4], ss[4:8], ss[8:12]
        asum = jnp.sum(jnp.sum(asum_ref[...], axis=0), axis=1, keepdims=True)
        ad = asum[0:4]
        ap = asum[4:8]
        ehd = ehd_ref[...]
        cnt = ehd[:, 511:512]
        bgat = bgat_ref[...]
        ehad = jnp.concatenate([ehd[:, 0:511], ad], axis=1)
        numer = (s1 + NB * bgat
                 + jnp.dot(s2 + cnt * bgat, wmsg_ref[...],
                           preferred_element_type=jnp.float32)
                 + jnp.dot(s3 + cnt * bgat, wmsgi_ref[...],
                           preferred_element_type=jnp.float32)
                 + jnp.dot(ehad, wge_ref[...],
                           preferred_element_type=jnp.float32)
                 + cnt * (bmsg_ref[...] + bmsgi_ref[...] + bge_ref[...]))
        out_nodes = numer / NB
        out_edges = jnp.concatenate([ehp_ref[:, 0:511], ap], axis=1) / EB
        gg = _lrelu(jnp.dot(y2_ref[...], wgp_ref[...],
                            preferred_element_type=jnp.float32) + bgp_ref[...], 0.01)
        pooled = jnp.concatenate([out_nodes, out_edges, gg], axis=1)
        o = _lrelu(jnp.dot(pooled, wf1_ref[...],
                           preferred_element_type=jnp.float32) + bf1_ref[...], 0.01)
        out_ref[...] = jax.nn.sigmoid(
            jnp.dot(o, wf2_ref[...], preferred_element_type=jnp.float32)
            + bf2_ref[...])


def _k6(a1_p, a2_p, a3_p, x_l, ehd, ehp, asum_p, y2, W_gp, b_gp, b_gat,
        W_msg, b_msg, W_msg_i, b_msg_i, W_ge, b_ge, W_f1, b_f1, W_f2, b_f2):
    BN = 2048
    nj = NPAD // BN
    full = lambda shape: pl.BlockSpec(shape, lambda j: tuple(0 for _ in shape))
    outs = pl.pallas_call(
        _k6_body,
        grid=(nj,),
        in_specs=[
            pl.BlockSpec((NW, B, BN), lambda j: (0, 0, j)),
            pl.BlockSpec((NW, B, BN), lambda j: (0, 0, j)),
            pl.BlockSpec((NW, B, BN), lambda j: (0, 0, j)),
            pl.BlockSpec((BN, 1024), lambda j: (j, 0)),
            full((B, 512)), full((B, 512)), full((NW, 8, 16)),
            full((B, 5)), full((5, 512)), full((1, 512)),
            full((1, 1024)), full((1024, 1024)), full((1, 1024)),
            full((1024, 1024)), full((1, 1024)),
            full((512, 1024)), full((1, 1024)),
            full((2048, 256)), full((1, 256)), full((256, 1)), full((1, 1)),
        ],
        out_specs=[
            full((12, 1024)),
            full((B, 1)),
        ],
        out_shape=[
            jax.ShapeDtypeStruct((12, 1024), jnp.float32),
            jax.ShapeDtypeStruct((B, 1), jnp.float32),
        ],
        compiler_params=pltpu.CompilerParams(
            dimension_semantics=("arbitrary",)),
    )(a1_p, a2_p, a3_p, x_l, ehd, ehp, asum_p, y2, W_gp, b_gp, b_gat,
      W_msg, b_msg, W_msg_i, b_msg_i, W_ge, b_ge, W_f1, b_f1, W_f2, b_f2)
    return outs[1]


def kernel(x, edge_index, edge_attr, y, batch, W_node, b_node, W_ep, b_ep,
           W_gp, b_gp, W_l, b_l, W_r, b_r, W_e, att, b_gat, W_msg, b_msg,
           W_msg_i, b_msg_i, W_ge, b_ge, W_f1, b_f1, W_f2, b_f2):
    f32 = jnp.float32
    loop = jnp.arange(N, dtype=edge_index.dtype)
    pad1 = jnp.zeros((EPAD - ETOT,), jnp.int32)
    src = jnp.concatenate([edge_index[0], loop, pad1])
    dst = jnp.concatenate([edge_index[1], loop, pad1])
    ea_pad = jnp.concatenate(
        [edge_attr, jnp.ones((N, 5), f32), jnp.zeros((EPAD - ETOT, 5), f32)], axis=0)
    ec = jnp.minimum(jnp.arange(EPAD, dtype=jnp.int32) // EB, B - 1)
    dst3 = dst.reshape(G, 1, BLK)
    ec3 = ec.reshape(G, 1, BLK)
    x2 = jnp.concatenate([x[:, None], jnp.zeros((NPAD - N, 1), f32)], axis=0)
    y2 = y.reshape(B, 5)

    r2 = lambda a: a.reshape(1, -1)

    x_l, xl_b, xr_b = _k1(x2, r2(W_node), r2(b_node), W_l, r2(b_l),
                          W_r, r2(b_r))
    as_i32 = lambda a: lax.bitcast_convert_type(
        a.reshape(NPAD, 512, 2), jnp.int32)
    as_bf = lambda a: lax.bitcast_convert_type(a, jnp.bfloat16).reshape(EPAD, 1024)
    gl_i, gr_i = _g1(as_i32(xl_b), as_i32(xr_b), src, dst)
    lg3, ehd, ehp, mxv = _k2(ea_pad, as_bf(gl_i), as_bf(gr_i), dst3, ec3,
                             W_ep, r2(b_ep), W_e.astype(jnp.bfloat16), r2(att))
    lg = lg3.reshape(EPAD)
    mx16 = mxv[0, :16]
    ex, den_p, deg_p, c_p = _k3(lg, src, dst, mx16)
    den1, deg1, c1 = _k4(den_p, deg_p, c_p)
    al, a1_p, a3_p, asum_p = _k5a(ex, src, dst, den1.reshape(NPAD),
                                  deg1.reshape(NPAD))
    a2_p = _k5b(al, src, dst, c1.reshape(4 * NPAD))
    out = _k6(a1_p.reshape(NW, B, NPAD), a2_p.reshape(NW, B, NPAD),
              a3_p.reshape(NW, B, NPAD), x_l, ehd, ehp,
              asum_p.reshape(NW, 8, 16), y2,
              W_gp, r2(b_gp), r2(b_gat), W_msg, r2(b_msg), W_msg_i,
              r2(b_msg_i), W_ge, r2(b_ge), W_f1, r2(b_f1), W_f2, r2(b_f2))
    return out


# speedup vs baseline: 5.4667x; 5.4667x over previous
import functools

import jax
import jax.numpy as jnp
from jax import lax
from jax.experimental import pallas as pl
from jax.experimental.pallas import tpu as pltpu
from jax.experimental.pallas import tpu_sc as plsc

N = 10000
B = 4
NB = N // B
E = 160000
ETOT = E + N
EB = ETOT // B
BLK = 1024
G = (ETOT + BLK - 1) // BLK
EPAD = G * BLK
NPAD = 10240
NW = 32
EW = EPAD // NW
NEG = -1.0e30


def _lrelu(v, s):
    return jnp.where(v >= 0, v, s * v)


def _pack(v):
    bits = lambda a: lax.bitcast_convert_type(
        a.astype(jnp.bfloat16).astype(jnp.float32), jnp.int32)
    lo = bits(v[:, 0:512])
    hi = bits(v[:, 512:1024])
    return lax.shift_right_logical(lo, 16) | (hi & jnp.int32(-65536))


def _unpack(w):
    f_lo = lax.bitcast_convert_type(w << 16, jnp.float32)
    f_hi = lax.bitcast_convert_type(w & jnp.int32(-65536), jnp.float32)
    return f_lo, f_hi


def _k1_body(x_ref, wn_ref, bn_ref, wl_ref, bl_ref, wr_ref, br_ref,
             xl_ref, xlp_ref, xrp_ref):
    h = _lrelu(x_ref[...] * wn_ref[...] + bn_ref[...], 0.01)
    xl = jnp.dot(h, wl_ref[...], preferred_element_type=jnp.float32) + bl_ref[...]
    xr = jnp.dot(h, wr_ref[...], preferred_element_type=jnp.float32) + br_ref[...]
    xl_ref[...] = xl
    xlp_ref[...] = _pack(xl)
    xrp_ref[...] = _pack(xr)


def _k1(x2, W_node, b_node, W_l, b_l, W_r, b_r):
    n_blk = NPAD // BLK
    full = lambda shape: pl.BlockSpec(shape, lambda g: (0, 0))
    return pl.pallas_call(
        _k1_body,
        grid=(n_blk,),
        in_specs=[
            pl.BlockSpec((BLK, 1), lambda g: (g, 0)),
            full((1, 512)), full((1, 512)),
            full((512, 1024)), full((1, 1024)),
            full((512, 1024)), full((1, 1024)),
        ],
        out_specs=[
            pl.BlockSpec((BLK, 1024), lambda g: (g, 0)),
            pl.BlockSpec((BLK, 512), lambda g: (g, 0)),
            pl.BlockSpec((BLK, 512), lambda g: (g, 0)),
        ],
        out_shape=[
            jax.ShapeDtypeStruct((NPAD, 1024), jnp.float32),
            jax.ShapeDtypeStruct((NPAD, 512), jnp.int32),
            jax.ShapeDtypeStruct((NPAD, 512), jnp.int32),
        ],
    )(x2, W_node, b_node, W_l, b_l, W_r, b_r)


def _g1_body(xl_hbm, xr_hbm, src_hbm, dst_hbm, gl_hbm, gr_hbm,
             src_v, dst_v, bufa, bufb, sema, semb):
    wid = lax.axis_index("s") * 2 + lax.axis_index("c")
    base = pl.multiple_of(wid * EW, 8)
    pltpu.sync_copy(src_hbm.at[pl.ds(base, EW)], src_v)
    pltpu.sync_copy(dst_hbm.at[pl.ds(base, EW)], dst_v)

    C = 64
    NFULL = EW // C

    def chunk(rows, off):
        da = pltpu.async_copy(xl_hbm.at[src_v.at[pl.ds(off, rows)]],
                              bufa.at[pl.ds(0, rows)], sema)
        db = pltpu.async_copy(xr_hbm.at[dst_v.at[pl.ds(off, rows)]],
                              bufb.at[pl.ds(0, rows)], semb)
        da.wait()
        db.wait()
        pltpu.sync_copy(bufa.at[pl.ds(0, rows)],
                        gl_hbm.at[pl.ds(base + off, rows)])
        pltpu.sync_copy(bufb.at[pl.ds(0, rows)],
                        gr_hbm.at[pl.ds(base + off, rows)])

    def body(k, _):
        chunk(C, k * C)
        return _

    lax.fori_loop(0, NFULL, body, None)
    chunk(EW - NFULL * C, NFULL * C)


def _g1(xl_i, xr_i, src, dst):
    mesh = plsc.VectorSubcoreMesh(core_axis_name="c", subcore_axis_name="s")
    f = pl.kernel(
        _g1_body,
        out_type=(
            jax.ShapeDtypeStruct((EPAD, 512), jnp.int32),
            jax.ShapeDtypeStruct((EPAD, 512), jnp.int32),
        ),
        mesh=mesh,
        compiler_params=pltpu.CompilerParams(needs_layout_passes=False),
        scratch_types=[
            pltpu.VMEM((EW,), jnp.int32),
            pltpu.VMEM((EW,), jnp.int32),
            pltpu.VMEM((64, 512), jnp.int32),
            pltpu.VMEM((64, 512), jnp.int32),
            pltpu.SemaphoreType.DMA,
            pltpu.SemaphoreType.DMA,
        ],
    )
    return f(xl_i, xr_i, src, dst)


def _k2_body(ea_ref, gl_ref, gr_ref, dst_ref, ec_ref, wep_ref, bep_ref,
             we_ref, att_ref, lg_ref, ehd_ref, ehp_ref, mx_ref):
    g = pl.program_id(0)
    eh = _lrelu(jnp.dot(ea_ref[...], wep_ref[...],
                        preferred_element_type=jnp.float32) + bep_ref[...], 0.01)
    z = jnp.dot(eh.astype(jnp.bfloat16), we_ref[...],
                preferred_element_type=jnp.float32)
    gl_lo, gl_hi = _unpack(gl_ref[...])
    gr_lo, gr_hi = _unpack(gr_ref[...])
    v_lo = z[:, 0:512] + gl_lo + gr_lo
    v_hi = z[:, 512:1024] + gl_hi + gr_hi
    att = att_ref[...]
    lg = (jnp.sum(_lrelu(v_lo, 0.2) * att[:, 0:512], axis=1, keepdims=True)
          + jnp.sum(_lrelu(v_hi, 0.2) * att[:, 512:1024], axis=1,
                    keepdims=True))

    col = jax.lax.broadcasted_iota(jnp.int32, (BLK, 1), 0) + g * BLK
    lgv = jnp.where(col < ETOT, lg, NEG)
    lg_ref[0] = lgv

    bm = jnp.max(lgv)
    prev = jnp.where(g == 0, NEG, mx_ref[...])
    mx_ref[...] = jnp.maximum(prev, jnp.full((1, 128), bm))

    row = jax.lax.broadcasted_iota(jnp.int32, (1, BLK), 1) + g * BLK
    valid2 = row < ETOT
    lanes = jax.lax.broadcasted_iota(jnp.int32, (B, BLK), 0)
    dstc = dst_ref[0] // NB
    oh_d = jnp.where((dstc == lanes) & valid2, 1.0, 0.0)
    oh_p = jnp.where((ec_ref[0] == lanes) & valid2, 1.0, 0.0)
    eh_aug = jnp.concatenate([eh, jnp.ones((BLK, 1), jnp.float32)], axis=1)

    d_contrib = jnp.dot(oh_d, eh_aug, preferred_element_type=jnp.float32)
    p_contrib = jnp.dot(oh_p, eh_aug, preferred_element_type=jnp.float32)
    ehd_ref[...] = jnp.where(g == 0, 0.0, ehd_ref[...]) + d_contrib
    ehp_ref[...] = jnp.where(g == 0, 0.0, ehp_ref[...]) + p_contrib


def _k2(ea_pad, gl, gr, dst3, ec3, W_ep, b_ep, W_e, att):
    full = lambda shape: pl.BlockSpec(shape, lambda g: (0, 0))
    return pl.pallas_call(
        _k2_body,
        grid=(G,),
        in_specs=[
            pl.BlockSpec((BLK, 5), lambda g: (g, 0)),
            pl.BlockSpec((BLK, 512), lambda g: (g, 0)),
            pl.BlockSpec((BLK, 512), lambda g: (g, 0)),

            pl.BlockSpec((1, 1, BLK), lambda g: (g, 0, 0)),
            pl.BlockSpec((1, 1, BLK), lambda g: (g, 0, 0)),
            full((5, 511)), full((1, 511)), full((511, 1024)), full((1, 1024)),
        ],
        out_specs=[
            pl.BlockSpec((1, BLK, 1), lambda g: (g, 0, 0)),
            full((B, 512)), full((B, 512)),
            pl.BlockSpec((1, 128), lambda g: (0, 0)),
        ],
        out_shape=[
            jax.ShapeDtypeStruct((G, BLK, 1), jnp.float32),
            jax.ShapeDtypeStruct((B, 512), jnp.float32),
            jax.ShapeDtypeStruct((B, 512), jnp.float32),
            jax.ShapeDtypeStruct((1, 128), jnp.float32),
        ],
        compiler_params=pltpu.CompilerParams(
            dimension_semantics=("arbitrary",)),
    )(ea_pad, gl, gr, dst3, ec3, W_ep, b_ep, W_e, att)


def _k3_body(lg_hbm, src_hbm, dst_hbm, mx_hbm,
             ex_hbm, den_hbm, deg_hbm, c_hbm,
             lgv, srcv, dstv, mxv, denv, degv, cv):
    wid = lax.axis_index("s") * 2 + lax.axis_index("c")
    base = pl.multiple_of(wid * EW, 8)
    pltpu.sync_copy(lg_hbm.at[pl.ds(base, EW)], lgv)
    pltpu.sync_copy(src_hbm.at[pl.ds(base, EW)], srcv)
    pltpu.sync_copy(dst_hbm.at[pl.ds(base, EW)], dstv)
    pltpu.sync_copy(mx_hbm, mxv)

    def zero(i, ref):
        ref[pl.ds(i * 16, 16)] = jnp.zeros((16,), jnp.float32)

    lax.fori_loop(0, NPAD // 16, lambda i, _: (zero(i, denv), _)[1], None)
    lax.fori_loop(0, NPAD // 16, lambda i, _: (zero(i, degv), _)[1], None)
    lax.fori_loop(0, 4 * NPAD // 16, lambda i, _: (zero(i, cv), _)[1], None)

    mx16 = mxv[...]
    lane = jnp.arange(16, dtype=jnp.int32)
    ones = jnp.ones((16,), jnp.float32)

    def body(i, _):
        sl = pl.ds(i * 16, 16)
        ex16 = jnp.exp(lgv[sl] - mx16)
        lgv[sl] = ex16
        d16 = dstv[sl]
        s16 = srcv[sl]
        valid = (base + i * 16 + lane) < ETOT
        plsc.addupdate_scatter(denv, [d16], ex16)
        plsc.addupdate_scatter(degv, [d16], ones, mask=valid)
        m16 = d16 // NB
        plsc.addupdate_scatter(cv, [m16 * NPAD + s16], ones, mask=valid)
        return _

    lax.fori_loop(0, EW // 16, body, None)

    pltpu.sync_copy(lgv, ex_hbm.at[pl.ds(base, EW)])
    pltpu.sync_copy(denv, den_hbm.at[wid])
    pltpu.sync_copy(degv, deg_hbm.at[wid])
    pltpu.sync_copy(cv, c_hbm.at[wid])


def _k3(lg, src, dst, mx16):
    mesh = plsc.VectorSubcoreMesh(core_axis_name="c", subcore_axis_name="s")
    f = pl.kernel(
        _k3_body,
        out_type=(
            jax.ShapeDtypeStruct((EPAD,), jnp.float32),
            jax.ShapeDtypeStruct((NW, NPAD), jnp.float32),
            jax.ShapeDtypeStruct((NW, NPAD), jnp.float32),
            jax.ShapeDtypeStruct((NW, 4 * NPAD), jnp.float32),
        ),
        mesh=mesh,
        compiler_params=pltpu.CompilerParams(needs_layout_passes=False),
        scratch_types=[
            pltpu.VMEM((EW,), jnp.float32),
            pltpu.VMEM((EW,), jnp.int32),
            pltpu.VMEM((EW,), jnp.int32),
            pltpu.VMEM((16,), jnp.float32),
            pltpu.VMEM((NPAD,), jnp.float32),
            pltpu.VMEM((NPAD,), jnp.float32),
            pltpu.VMEM((4 * NPAD,), jnp.float32),
        ],
    )
    return f(lg, src, dst, mx16)


def _k4_body(denp_ref, degp_ref, cp_ref, den_ref, deg_ref, c_ref):
    den_ref[...] = jnp.sum(denp_ref[...], axis=0, keepdims=True)
    deg_ref[...] = jnp.sum(degp_ref[...], axis=0, keepdims=True)
    c_ref[...] = jnp.sum(cp_ref[...], axis=0, keepdims=True)


def _k4(den_p, deg_p, c_p):
    return pl.pallas_call(
        _k4_body,
        out_shape=[
            jax.ShapeDtypeStruct((1, NPAD), jnp.float32),
            jax.ShapeDtypeStruct((1, NPAD), jnp.float32),
            jax.ShapeDtypeStruct((1, 4 * NPAD), jnp.float32),
        ],
    )(den_p, deg_p, c_p)


def _k5a_body(ex_hbm, src_hbm, dst_hbm, den_hbm, deg_hbm,
              al_hbm, a1_hbm, a3_hbm, asum_hbm,
              exv, srcv, dstv, denv, degv, a1v, a3v, asumv):
    wid = lax.axis_index("s") * 2 + lax.axis_index("c")
    base = pl.multiple_of(wid * EW, 8)
    pltpu.sync_copy(ex_hbm.at[pl.ds(base, EW)], exv)
    pltpu.sync_copy(src_hbm.at[pl.ds(base, EW)], srcv)
    pltpu.sync_copy(dst_hbm.at[pl.ds(base, EW)], dstv)
    pltpu.sync_copy(den_hbm, denv)
    pltpu.sync_copy(deg_hbm, degv)

    def zero(i, ref):
        ref[pl.ds(i * 16, 16)] = jnp.zeros((16,), jnp.float32)

    lax.fori_loop(0, 4 * NPAD // 16, lambda i, _: (zero(i, a1v), _)[1], None)
    lax.fori_loop(0, 4 * NPAD // 16, lambda i, _: (zero(i, a3v), _)[1], None)
    for b in range(8):
        asumv[pl.ds(b * 16, 16)] = jnp.zeros((16,), jnp.float32)

    lane = jnp.arange(16, dtype=jnp.int32)
    zf = jnp.zeros((16,), jnp.float32)

    def body(i, _):
        sl = pl.ds(i * 16, 16)
        d16 = dstv[sl]
        s16 = srcv[sl]
        den16 = plsc.load_gather(denv, [d16])
        a16 = exv[sl] / den16
        exv[sl] = a16
        deg16 = plsc.load_gather(degv, [d16])
        m16 = d16 // NB
        fl = m16 * NPAD + s16
        plsc.addupdate_scatter(a1v, [fl], a16)
        plsc.addupdate_scatter(a3v, [fl], deg16 * a16)
        p16 = (base + i * 16 + lane) // EB
        for b in range(B):
            plsc.addupdate(asumv.at[pl.ds(b * 16, 16)],
                           jnp.where(m16 == b, a16, zf))
            plsc.addupdate(asumv.at[pl.ds((4 + b) * 16, 16)],
                           jnp.where(p16 == b, a16, zf))
        return _

    lax.fori_loop(0, EW // 16, body, None)

    pltpu.sync_copy(exv, al_hbm.at[pl.ds(base, EW)])
    pltpu.sync_copy(a1v, a1_hbm.at[wid])
    pltpu.sync_copy(a3v, a3_hbm.at[wid])
    pltpu.sync_copy(asumv, asum_hbm.at[wid])


def _k5a(ex, src, dst, den1, deg1):
    mesh = plsc.VectorSubcoreMesh(core_axis_name="c", subcore_axis_name="s")
    f = pl.kernel(
        _k5a_body,
        out_type=(
            jax.ShapeDtypeStruct((EPAD,), jnp.float32),
            jax.ShapeDtypeStruct((NW, 4 * NPAD), jnp.float32),
            jax.ShapeDtypeStruct((NW, 4 * NPAD), jnp.float32),
            jax.ShapeDtypeStruct((NW, 128), jnp.float32),
        ),
        mesh=mesh,
        compiler_params=pltpu.CompilerParams(needs_layout_passes=False),
        scratch_types=[
            pltpu.VMEM((EW,), jnp.float32),
            pltpu.VMEM((EW,), jnp.int32),
            pltpu.VMEM((EW,), jnp.int32),
            pltpu.VMEM((NPAD,), jnp.float32),
            pltpu.VMEM((NPAD,), jnp.float32),
            pltpu.VMEM((4 * NPAD,), jnp.float32),
            pltpu.VMEM((4 * NPAD,), jnp.float32),
            pltpu.VMEM((128,), jnp.float32),
        ],
    )
    return f(ex, src, dst, den1, deg1)


def _k5b_body(al_hbm, src_hbm, dst_hbm, c_hbm, a2_hbm,
              alv, srcv, dstv, cv, a2v):
    wid = lax.axis_index("s") * 2 + lax.axis_index("c")
    base = pl.multiple_of(wid * EW, 8)
    pltpu.sync_copy(al_hbm.at[pl.ds(base, EW)], alv)
    pltpu.sync_copy(src_hbm.at[pl.ds(base, EW)], srcv)
    pltpu.sync_copy(dst_hbm.at[pl.ds(base, EW)], dstv)
    pltpu.sync_copy(c_hbm, cv)

    def zero(i, ref):
        ref[pl.ds(i * 16, 16)] = jnp.zeros((16,), jnp.float32)

    lax.fori_loop(0, 4 * NPAD // 16, lambda i, _: (zero(i, a2v), _)[1], None)

    def body(i, _):
        sl = pl.ds(i * 16, 16)
        a16 = alv[sl]
        d16 = dstv[sl]
        s16 = srcv[sl]
        for b in range(B):
            cb = plsc.load_gather(cv, [b * NPAD + d16])
            plsc.addupdate_scatter(a2v, [b * NPAD + s16], cb * a16)
        return _

    lax.fori_loop(0, EW // 16, body, None)
    pltpu.sync_copy(a2v, a2_hbm.at[wid])


def _k5b(al, src, dst, c1):
    mesh = plsc.VectorSubcoreMesh(core_axis_name="c", subcore_axis_name="s")
    f = pl.kernel(
        _k5b_body,
        out_type=jax.ShapeDtypeStruct((NW, 4 * NPAD), jnp.float32),
        mesh=mesh,
        compiler_params=pltpu.CompilerParams(needs_layout_passes=False),
        scratch_types=[
            pltpu.VMEM((EW,), jnp.float32),
            pltpu.VMEM((EW,), jnp.int32),
            pltpu.VMEM((EW,), jnp.int32),
            pltpu.VMEM((4 * NPAD,), jnp.float32),
            pltpu.VMEM((4 * NPAD,), jnp.float32),
        ],
    )
    return f(al, src, dst, c1)


def _k6_body(a1_ref, a2_ref, a3_ref, xl_ref, ehd_ref, ehp_ref, asum_ref,
             y2_ref, wgp_ref, bgp_ref, bgat_ref, wmsg_ref, bmsg_ref,
             wmsgi_ref, bmsgi_ref, wge_ref, bge_ref, wf1_ref, bf1_ref,
             wf2_ref, bf2_ref, ss_ref, out_ref):
    j = pl.program_id(0)
    a1 = jnp.sum(a1_ref[...], axis=0)
    a2 = jnp.sum(a2_ref[...], axis=0)
    a3 = jnp.sum(a3_ref[...], axis=0)
    astack = jnp.concatenate([a1, a2, a3], axis=0)
    contrib = jnp.dot(astack, xl_ref[...], preferred_element_type=jnp.float32)
    ss_ref[...] = jnp.where(j == 0, 0.0, ss_ref[...]) + contrib

    @pl.when(j == pl.num_programs(0) - 1)
    def _():
        ss = ss_ref[...]
        s1, s2, s3 = ss[0:4], ss[4:8], ss[8:12]
        asum = jnp.sum(jnp.sum(asum_ref[...], axis=0), axis=1, keepdims=True)
        ad = asum[0:4]
        ap = asum[4:8]
        ehd = ehd_ref[...]
        cnt = ehd[:, 511:512]
        bgat = bgat_ref[...]
        ehad = jnp.concatenate([ehd[:, 0:511], ad], axis=1)
        numer = (s1 + NB * bgat
                 + jnp.dot(s2 + cnt * bgat, wmsg_ref[...],
                           preferred_element_type=jnp.float32)
                 + jnp.dot(s3 + cnt * bgat, wmsgi_ref[...],
                           preferred_element_type=jnp.float32)
                 + jnp.dot(ehad, wge_ref[...],
                           preferred_element_type=jnp.float32)
                 + cnt * (bmsg_ref[...] + bmsgi_ref[...] + bge_ref[...]))
        out_nodes = numer / NB
        out_edges = jnp.concatenate([ehp_ref[:, 0:511], ap], axis=1) / EB
        gg = _lrelu(jnp.dot(y2_ref[...], wgp_ref[...],
                            preferred_element_type=jnp.float32) + bgp_ref[...], 0.01)
        pooled = jnp.concatenate([out_nodes, out_edges, gg], axis=1)
        o = _lrelu(jnp.dot(pooled, wf1_ref[...],
                           preferred_element_type=jnp.float32) + bf1_ref[...], 0.01)
        out_ref[...] = jax.nn.sigmoid(
            jnp.dot(o, wf2_ref[...], preferred_element_type=jnp.float32)
            + bf2_ref[...])


def _k6(a1_p, a2_p, a3_p, x_l, ehd, ehp, asum_p, y2, W_gp, b_gp, b_gat,
        W_msg, b_msg, W_msg_i, b_msg_i, W_ge, b_ge, W_f1, b_f1, W_f2, b_f2):
    BN = 2048
    nj = NPAD // BN
    full = lambda shape: pl.BlockSpec(shape, lambda j: tuple(0 for _ in shape))
    outs = pl.pallas_call(
        _k6_body,
        grid=(nj,),
        in_specs=[
            pl.BlockSpec((NW, B, BN), lambda j: (0, 0, j)),
            pl.BlockSpec((NW, B, BN), lambda j: (0, 0, j)),
            pl.BlockSpec((NW, B, BN), lambda j: (0, 0, j)),
            pl.BlockSpec((BN, 1024), lambda j: (j, 0)),
            full((B, 512)), full((B, 512)), full((NW, 8, 16)),
            full((B, 5)), full((5, 512)), full((1, 512)),
            full((1, 1024)), full((1024, 1024)), full((1, 1024)),
            full((1024, 1024)), full((1, 1024)),
            full((512, 1024)), full((1, 1024)),
            full((2048, 256)), full((1, 256)), full((256, 1)), full((1, 1)),
        ],
        out_specs=[
            full((12, 1024)),
            full((B, 1)),
        ],
        out_shape=[
            jax.ShapeDtypeStruct((12, 1024), jnp.float32),
            jax.ShapeDtypeStruct((B, 1), jnp.float32),
        ],
        compiler_params=pltpu.CompilerParams(
            dimension_semantics=("arbitrary",)),
    )(a1_p, a2_p, a3_p, x_l, ehd, ehp, asum_p, y2, W_gp, b_gp, b_gat,
      W_msg, b_msg, W_msg_i, b_msg_i, W_ge, b_ge, W_f1, b_f1, W_f2, b_f2)
    return outs[1]


def kernel(x, edge_index, edge_attr, y, batch, W_node, b_node, W_ep, b_ep,
           W_gp, b_gp, W_l, b_l, W_r, b_r, W_e, att, b_gat, W_msg, b_msg,
           W_msg_i, b_msg_i, W_ge, b_ge, W_f1, b_f1, W_f2, b_f2):
    f32 = jnp.float32
    loop = jnp.arange(N, dtype=edge_index.dtype)
    pad1 = jnp.zeros((EPAD - ETOT,), jnp.int32)
    src = jnp.concatenate([edge_index[0], loop, pad1])
    dst = jnp.concatenate([edge_index[1], loop, pad1])
    ea_pad = jnp.concatenate(
        [edge_attr, jnp.ones((N, 5), f32), jnp.zeros((EPAD - ETOT, 5), f32)], axis=0)
    ec = jnp.minimum(jnp.arange(EPAD, dtype=jnp.int32) // EB, B - 1)
    dst3 = dst.reshape(G, 1, BLK)
    ec3 = ec.reshape(G, 1, BLK)
    x2 = jnp.concatenate([x[:, None], jnp.zeros((NPAD - N, 1), f32)], axis=0)
    y2 = y.reshape(B, 5)

    r2 = lambda a: a.reshape(1, -1)

    x_l, xl_p, xr_p = _k1(x2, r2(W_node), r2(b_node), W_l, r2(b_l),
                          W_r, r2(b_r))
    gl_i, gr_i = _g1(xl_p, xr_p, src, dst)
    lg3, ehd, ehp, mxv = _k2(ea_pad, gl_i, gr_i, dst3, ec3,
                             W_ep, r2(b_ep), W_e.astype(jnp.bfloat16), r2(att))
    lg = lg3.reshape(EPAD)
    mx16 = mxv[0, :16]
    ex, den_p, deg_p, c_p = _k3(lg, src, dst, mx16)
    den1, deg1, c1 = _k4(den_p, deg_p, c_p)
    al, a1_p, a3_p, asum_p = _k5a(ex, src, dst, den1.reshape(NPAD),
                                  deg1.reshape(NPAD))
    a2_p = _k5b(al, src, dst, c1.reshape(4 * NPAD))
    out = _k6(a1_p.reshape(NW, B, NPAD), a2_p.reshape(NW, B, NPAD),
              a3_p.reshape(NW, B, NPAD), x_l, ehd, ehp,
              asum_p.reshape(NW, 8, 16), y2,
              W_gp, r2(b_gp), r2(b_gat), W_msg, r2(b_msg), W_msg_i,
              r2(b_msg_i), W_ge, r2(b_ge), W_f1, r2(b_f1), W_f2, r2(b_f2))
    return out


# speedup vs baseline: 5.5678x; 1.0185x over previous
import functools

import jax
import jax.numpy as jnp
from jax import lax
from jax.experimental import pallas as pl
from jax.experimental.pallas import tpu as pltpu
from jax.experimental.pallas import tpu_sc as plsc

N = 10000
B = 4
NB = N // B
E = 160000
ETOT = E + N
EB = ETOT // B
BLK = 1024
G = (ETOT + BLK - 1) // BLK
EPAD = G * BLK
NPAD = 10240
NW = 32
EW = EPAD // NW
NEG = -1.0e30


def _lrelu(v, s):
    return jnp.where(v >= 0, v, s * v)


def _pack(v):
    bits = lambda a: lax.bitcast_convert_type(
        a.astype(jnp.bfloat16).astype(jnp.float32), jnp.int32)
    lo = bits(v[:, 0:512])
    hi = bits(v[:, 512:1024])
    return lax.shift_right_logical(lo, 16) | (hi & jnp.int32(-65536))


def _unpack(w):
    f_lo = lax.bitcast_convert_type(w << 16, jnp.float32)
    f_hi = lax.bitcast_convert_type(w & jnp.int32(-65536), jnp.float32)
    return f_lo, f_hi


def _k1_body(x_ref, wn_ref, bn_ref, wl_ref, bl_ref, wr_ref, br_ref,
             xl_ref, xlp_ref, xrp_ref):
    h = _lrelu(x_ref[...] * wn_ref[...] + bn_ref[...], 0.01).astype(jnp.bfloat16)
    xl = jnp.dot(h, wl_ref[...], preferred_element_type=jnp.float32) + bl_ref[...]
    xr = jnp.dot(h, wr_ref[...], preferred_element_type=jnp.float32) + br_ref[...]
    xl_ref[...] = xl
    xlp_ref[...] = _pack(xl)
    xrp_ref[...] = _pack(xr)


def _k1(x2, W_node, b_node, W_l, b_l, W_r, b_r):
    n_blk = NPAD // BLK
    full = lambda shape: pl.BlockSpec(shape, lambda g: (0, 0))
    return pl.pallas_call(
        _k1_body,
        grid=(n_blk,),
        in_specs=[
            pl.BlockSpec((BLK, 1), lambda g: (g, 0)),
            full((1, 512)), full((1, 512)),
            full((512, 1024)), full((1, 1024)),
            full((512, 1024)), full((1, 1024)),
        ],
        out_specs=[
            pl.BlockSpec((BLK, 1024), lambda g: (g, 0)),
            pl.BlockSpec((BLK, 512), lambda g: (g, 0)),
            pl.BlockSpec((BLK, 512), lambda g: (g, 0)),
        ],
        out_shape=[
            jax.ShapeDtypeStruct((NPAD, 1024), jnp.float32),
            jax.ShapeDtypeStruct((NPAD, 512), jnp.int32),
            jax.ShapeDtypeStruct((NPAD, 512), jnp.int32),
        ],
    )(x2, W_node, b_node, W_l, b_l, W_r, b_r)


GC = 48
GNCH = EW // GC
GTAIL = EW - GNCH * GC
GPP = (GNCH - 1) // 2 * 2


def _g1_body(xl_hbm, xr_hbm, src_hbm, dst_hbm, gl_hbm, gr_hbm,
             src_v, dst_v, bufa, bufb, sga, sgb, swa0, swa1, swb0, swb1):
    wid = lax.axis_index("s") * 2 + lax.axis_index("c")
    base = pl.multiple_of(wid * EW, 8)
    pltpu.sync_copy(src_hbm.at[pl.ds(base, EW)], src_v)
    pltpu.sync_copy(dst_hbm.at[pl.ds(base, EW)], dst_v)
    swa = (swa0, swa1)
    swb = (swb0, swb1)

    def gather(k, p, rows):
        da = pltpu.async_copy(xl_hbm.at[src_v.at[pl.ds(k * GC, rows)]],
                              bufa.at[p].at[pl.ds(0, rows)], sga)
        db = pltpu.async_copy(xr_hbm.at[dst_v.at[pl.ds(k * GC, rows)]],
                              bufb.at[p].at[pl.ds(0, rows)], sgb)
        da.wait()
        db.wait()

    def wr(k, p, rows, wait):
        dsta = gl_hbm.at[pl.ds(base + k * GC, rows)]
        dstb = gr_hbm.at[pl.ds(base + k * GC, rows)]
        if wait:
            pltpu.sync_copy(bufa.at[p].at[pl.ds(0, rows)], dsta)
            pltpu.sync_copy(bufb.at[p].at[pl.ds(0, rows)], dstb)
        else:
            pltpu.async_copy(bufa.at[p].at[pl.ds(0, rows)], dsta, swa[p])
            pltpu.async_copy(bufb.at[p].at[pl.ds(0, rows)], dstb, swb[p])

    def drain(p):
        pltpu.make_async_copy(bufa.at[p], gl_hbm.at[pl.ds(base, GC)],
                              swa[p]).wait()
        pltpu.make_async_copy(bufb.at[p], gr_hbm.at[pl.ds(base, GC)],
                              swb[p]).wait()

    def body(o, _):
        for p in range(2):
            k = o * 2 + p

            @pl.when(o > 0)
            def _():
                drain(p)

            gather(k, p, GC)
            wr(k, p, GC, wait=False)
        return _

    lax.fori_loop(0, GPP // 2, body, None)
    drain(0)
    drain(1)
    for k in range(GPP, GNCH):
        gather(k, 0, GC)
        wr(k, 0, GC, wait=True)
    if GTAIL:
        gather(GNCH, 0, GTAIL)
        wr(GNCH, 0, GTAIL, wait=True)


def _g1(xl_i, xr_i, src, dst):
    mesh = plsc.VectorSubcoreMesh(core_axis_name="c", subcore_axis_name="s")
    f = pl.kernel(
        _g1_body,
        out_type=(
            jax.ShapeDtypeStruct((EPAD, 512), jnp.int32),
            jax.ShapeDtypeStruct((EPAD, 512), jnp.int32),
        ),
        mesh=mesh,
        compiler_params=pltpu.CompilerParams(needs_layout_passes=False),
        scratch_types=[
            pltpu.VMEM((EW,), jnp.int32),
            pltpu.VMEM((EW,), jnp.int32),
            pltpu.VMEM((2, GC, 512), jnp.int32),
            pltpu.VMEM((2, GC, 512), jnp.int32),
            pltpu.SemaphoreType.DMA,
            pltpu.SemaphoreType.DMA,
            pltpu.SemaphoreType.DMA,
            pltpu.SemaphoreType.DMA,
            pltpu.SemaphoreType.DMA,
            pltpu.SemaphoreType.DMA,
        ],
    )
    return f(xl_i, xr_i, src, dst)


def _k2_body(ea_ref, gl_ref, gr_ref, dst_ref, ec_ref, wep_ref, bep_ref,
             we_ref, att_ref, lg_ref, ehd_ref, ehp_ref, mx_ref):
    g = pl.program_id(0)
    eh = _lrelu(jnp.dot(ea_ref[...], wep_ref[...],
                        preferred_element_type=jnp.float32) + bep_ref[...], 0.01)
    z = jnp.dot(eh.astype(jnp.bfloat16), we_ref[...],
                preferred_element_type=jnp.float32)
    gl_lo, gl_hi = _unpack(gl_ref[...])
    gr_lo, gr_hi = _unpack(gr_ref[...])
    v_lo = z[:, 0:512] + gl_lo + gr_lo
    v_hi = z[:, 512:1024] + gl_hi + gr_hi
    att = att_ref[...]
    lg = (jnp.sum(_lrelu(v_lo, 0.2) * att[:, 0:512], axis=1, keepdims=True)
          + jnp.sum(_lrelu(v_hi, 0.2) * att[:, 512:1024], axis=1,
                    keepdims=True))

    col = jax.lax.broadcasted_iota(jnp.int32, (BLK, 1), 0) + g * BLK
    lgv = jnp.where(col < ETOT, lg, NEG)
    lg_ref[0] = lgv

    bm = jnp.max(lgv)
    prev = jnp.where(g == 0, NEG, mx_ref[...])
    mx_ref[...] = jnp.maximum(prev, jnp.full((1, 128), bm))

    row = jax.lax.broadcasted_iota(jnp.int32, (1, BLK), 1) + g * BLK
    valid2 = row < ETOT
    lanes = jax.lax.broadcasted_iota(jnp.int32, (B, BLK), 0)
    dstc = dst_ref[0] // NB
    oh_d = jnp.where((dstc == lanes) & valid2, 1.0, 0.0)
    oh_p = jnp.where((ec_ref[0] == lanes) & valid2, 1.0, 0.0)
    eh_aug = jnp.concatenate([eh, jnp.ones((BLK, 1), jnp.float32)], axis=1)

    d_contrib = jnp.dot(oh_d, eh_aug, preferred_element_type=jnp.float32)
    p_contrib = jnp.dot(oh_p, eh_aug, preferred_element_type=jnp.float32)
    ehd_ref[...] = jnp.where(g == 0, 0.0, ehd_ref[...]) + d_contrib
    ehp_ref[...] = jnp.where(g == 0, 0.0, ehp_ref[...]) + p_contrib


def _k2(ea_pad, gl, gr, dst3, ec3, W_ep, b_ep, W_e, att):
    full = lambda shape: pl.BlockSpec(shape, lambda g: (0, 0))
    return pl.pallas_call(
        _k2_body,
        grid=(G,),
        in_specs=[
            pl.BlockSpec((BLK, 5), lambda g: (g, 0)),
            pl.BlockSpec((BLK, 512), lambda g: (g, 0)),
            pl.BlockSpec((BLK, 512), lambda g: (g, 0)),

            pl.BlockSpec((1, 1, BLK), lambda g: (g, 0, 0)),
            pl.BlockSpec((1, 1, BLK), lambda g: (g, 0, 0)),
            full((5, 511)), full((1, 511)), full((511, 1024)), full((1, 1024)),
        ],
        out_specs=[
            pl.BlockSpec((1, BLK, 1), lambda g: (g, 0, 0)),
            full((B, 512)), full((B, 512)),
            pl.BlockSpec((1, 128), lambda g: (0, 0)),
        ],
        out_shape=[
            jax.ShapeDtypeStruct((G, BLK, 1), jnp.float32),
            jax.ShapeDtypeStruct((B, 512), jnp.float32),
            jax.ShapeDtypeStruct((B, 512), jnp.float32),
            jax.ShapeDtypeStruct((1, 128), jnp.float32),
        ],
        compiler_params=pltpu.CompilerParams(
            dimension_semantics=("arbitrary",)),
    )(ea_pad, gl, gr, dst3, ec3, W_ep, b_ep, W_e, att)


def _k3_body(lg_hbm, src_hbm, dst_hbm, mx_hbm,
             ex_hbm, den_hbm, deg_hbm, c_hbm,
             lgv, srcv, dstv, mxv, denv, degv, cv):
    wid = lax.axis_index("s") * 2 + lax.axis_index("c")
    base = pl.multiple_of(wid * EW, 8)
    pltpu.sync_copy(lg_hbm.at[pl.ds(base, EW)], lgv)
    pltpu.sync_copy(src_hbm.at[pl.ds(base, EW)], srcv)
    pltpu.sync_copy(dst_hbm.at[pl.ds(base, EW)], dstv)
    pltpu.sync_copy(mx_hbm, mxv)

    def zero(i, ref):
        ref[pl.ds(i * 16, 16)] = jnp.zeros((16,), jnp.float32)

    lax.fori_loop(0, NPAD // 16, lambda i, _: (zero(i, denv), _)[1], None)
    lax.fori_loop(0, NPAD // 16, lambda i, _: (zero(i, degv), _)[1], None)
    lax.fori_loop(0, 4 * NPAD // 16, lambda i, _: (zero(i, cv), _)[1], None)

    mx16 = mxv[...]
    lane = jnp.arange(16, dtype=jnp.int32)
    ones = jnp.ones((16,), jnp.float32)

    def body(i, _):
        sl = pl.ds(i * 16, 16)
        ex16 = jnp.exp(lgv[sl] - mx16)
        lgv[sl] = ex16
        d16 = dstv[sl]
        s16 = srcv[sl]
        valid = (base + i * 16 + lane) < ETOT
        plsc.addupdate_scatter(denv, [d16], ex16)
        plsc.addupdate_scatter(degv, [d16], ones, mask=valid)
        m16 = d16 // NB
        plsc.addupdate_scatter(cv, [m16 * NPAD + s16], ones, mask=valid)
        return _

    lax.fori_loop(0, EW // 16, body, None)

    pltpu.sync_copy(lgv, ex_hbm.at[pl.ds(base, EW)])
    pltpu.sync_copy(denv, den_hbm.at[wid])
    pltpu.sync_copy(degv, deg_hbm.at[wid])
    pltpu.sync_copy(cv, c_hbm.at[wid])


def _k3(lg, src, dst, mx16):
    mesh = plsc.VectorSubcoreMesh(core_axis_name="c", subcore_axis_name="s")
    f = pl.kernel(
        _k3_body,
        out_type=(
            jax.ShapeDtypeStruct((EPAD,), jnp.float32),
            jax.ShapeDtypeStruct((NW, NPAD), jnp.float32),
            jax.ShapeDtypeStruct((NW, NPAD), jnp.float32),
            jax.ShapeDtypeStruct((NW, 4 * NPAD), jnp.float32),
        ),
        mesh=mesh,
        compiler_params=pltpu.CompilerParams(needs_layout_passes=False),
        scratch_types=[
            pltpu.VMEM((EW,), jnp.float32),
            pltpu.VMEM((EW,), jnp.int32),
            pltpu.VMEM((EW,), jnp.int32),
            pltpu.VMEM((16,), jnp.float32),
            pltpu.VMEM((NPAD,), jnp.float32),
            pltpu.VMEM((NPAD,), jnp.float32),
            pltpu.VMEM((4 * NPAD,), jnp.float32),
        ],
    )
    return f(lg, src, dst, mx16)


def _k4_body(denp_ref, degp_ref, cp_ref, den_ref, deg_ref, c_ref):
    den_ref[...] = jnp.sum(denp_ref[...], axis=0, keepdims=True)
    deg_ref[...] = jnp.sum(degp_ref[...], axis=0, keepdims=True)
    c_ref[...] = jnp.sum(cp_ref[...], axis=0, keepdims=True)


def _k4(den_p, deg_p, c_p):
    return pl.pallas_call(
        _k4_body,
        out_shape=[
            jax.ShapeDtypeStruct((1, NPAD), jnp.float32),
            jax.ShapeDtypeStruct((1, NPAD), jnp.float32),
            jax.ShapeDtypeStruct((1, 4 * NPAD), jnp.float32),
        ],
    )(den_p, deg_p, c_p)


def _k5a_body(ex_hbm, src_hbm, dst_hbm, den_hbm, deg_hbm,
              al_hbm, a1_hbm, a3_hbm, asum_hbm,
              exv, srcv, dstv, denv, degv, a1v, a3v, asumv):
    wid = lax.axis_index("s") * 2 + lax.axis_index("c")
    base = pl.multiple_of(wid * EW, 8)
    pltpu.sync_copy(ex_hbm.at[pl.ds(base, EW)], exv)
    pltpu.sync_copy(src_hbm.at[pl.ds(base, EW)], srcv)
    pltpu.sync_copy(dst_hbm.at[pl.ds(base, EW)], dstv)
    pltpu.sync_copy(den_hbm, denv)
    pltpu.sync_copy(deg_hbm, degv)

    def zero(i, ref):
        ref[pl.ds(i * 16, 16)] = jnp.zeros((16,), jnp.float32)

    lax.fori_loop(0, 4 * NPAD // 16, lambda i, _: (zero(i, a1v), _)[1], None)
    lax.fori_loop(0, 4 * NPAD // 16, lambda i, _: (zero(i, a3v), _)[1], None)
    for b in range(8):
        asumv[pl.ds(b * 16, 16)] = jnp.zeros((16,), jnp.float32)

    lane = jnp.arange(16, dtype=jnp.int32)
    zf = jnp.zeros((16,), jnp.float32)

    def body(i, _):
        sl = pl.ds(i * 16, 16)
        d16 = dstv[sl]
        s16 = srcv[sl]
        den16 = plsc.load_gather(denv, [d16])
        a16 = exv[sl] / den16
        exv[sl] = a16
        deg16 = plsc.load_gather(degv, [d16])
        m16 = d16 // NB
        fl = m16 * NPAD + s16
        plsc.addupdate_scatter(a1v, [fl], a16)
        plsc.addupdate_scatter(a3v, [fl], deg16 * a16)
        p16 = (base + i * 16 + lane) // EB
        for b in range(B):
            plsc.addupdate(asumv.at[pl.ds(b * 16, 16)],
                           jnp.where(m16 == b, a16, zf))
            plsc.addupdate(asumv.at[pl.ds((4 + b) * 16, 16)],
                           jnp.where(p16 == b, a16, zf))
        return _

    lax.fori_loop(0, EW // 16, body, None)

    pltpu.sync_copy(exv, al_hbm.at[pl.ds(base, EW)])
    pltpu.sync_copy(a1v, a1_hbm.at[wid])
    pltpu.sync_copy(a3v, a3_hbm.at[wid])
    pltpu.sync_copy(asumv, asum_hbm.at[wid])


def _k5a(ex, src, dst, den1, deg1):
    mesh = plsc.VectorSubcoreMesh(core_axis_name="c", subcore_axis_name="s")
    f = pl.kernel(
        _k5a_body,
        out_type=(
            jax.ShapeDtypeStruct((EPAD,), jnp.float32),
            jax.ShapeDtypeStruct((NW, 4 * NPAD), jnp.float32),
            jax.ShapeDtypeStruct((NW, 4 * NPAD), jnp.float32),
            jax.ShapeDtypeStruct((NW, 128), jnp.float32),
        ),
        mesh=mesh,
        compiler_params=pltpu.CompilerParams(needs_layout_passes=False),
        scratch_types=[
            pltpu.VMEM((EW,), jnp.float32),
            pltpu.VMEM((EW,), jnp.int32),
            pltpu.VMEM((EW,), jnp.int32),
            pltpu.VMEM((NPAD,), jnp.float32),
            pltpu.VMEM((NPAD,), jnp.float32),
            pltpu.VMEM((4 * NPAD,), jnp.float32),
            pltpu.VMEM((4 * NPAD,), jnp.float32),
            pltpu.VMEM((128,), jnp.float32),
        ],
    )
    return f(ex, src, dst, den1, deg1)


def _k5b_body(al_hbm, src_hbm, dst_hbm, c_hbm, a2_hbm,
              alv, srcv, dstv, cv, a2v):
    wid = lax.axis_index("s") * 2 + lax.axis_index("c")
    base = pl.multiple_of(wid * EW, 8)
    pltpu.sync_copy(al_hbm.at[pl.ds(base, EW)], alv)
    pltpu.sync_copy(src_hbm.at[pl.ds(base, EW)], srcv)
    pltpu.sync_copy(dst_hbm.at[pl.ds(base, EW)], dstv)
    pltpu.sync_copy(c_hbm, cv)

    def zero(i, ref):
        ref[pl.ds(i * 16, 16)] = jnp.zeros((16,), jnp.float32)

    lax.fori_loop(0, 4 * NPAD // 16, lambda i, _: (zero(i, a2v), _)[1], None)

    def body(i, _):
        sl = pl.ds(i * 16, 16)
        a16 = alv[sl]
        d16 = dstv[sl]
        s16 = srcv[sl]
        for b in range(B):
            cb = plsc.load_gather(cv, [b * NPAD + d16])
            plsc.addupdate_scatter(a2v, [b * NPAD + s16], cb * a16)
        return _

    lax.fori_loop(0, EW // 16, body, None)
    pltpu.sync_copy(a2v, a2_hbm.at[wid])


def _k5b(al, src, dst, c1):
    mesh = plsc.VectorSubcoreMesh(core_axis_name="c", subcore_axis_name="s")
    f = pl.kernel(
        _k5b_body,
        out_type=jax.ShapeDtypeStruct((NW, 4 * NPAD), jnp.float32),
        mesh=mesh,
        compiler_params=pltpu.CompilerParams(needs_layout_passes=False),
        scratch_types=[
            pltpu.VMEM((EW,), jnp.float32),
            pltpu.VMEM((EW,), jnp.int32),
            pltpu.VMEM((EW,), jnp.int32),
            pltpu.VMEM((4 * NPAD,), jnp.float32),
            pltpu.VMEM((4 * NPAD,), jnp.float32),
        ],
    )
    return f(al, src, dst, c1)


def _k6_body(a1_ref, a2_ref, a3_ref, xl_ref, ehd_ref, ehp_ref, asum_ref,
             y2_ref, wgp_ref, bgp_ref, bgat_ref, wmsg_ref, bmsg_ref,
             wmsgi_ref, bmsgi_ref, wge_ref, bge_ref, wf1_ref, bf1_ref,
             wf2_ref, bf2_ref, ss_ref, out_ref):
    j = pl.program_id(0)
    a1 = jnp.sum(a1_ref[...], axis=0)
    a2 = jnp.sum(a2_ref[...], axis=0)
    a3 = jnp.sum(a3_ref[...], axis=0)
    astack = jnp.concatenate([a1, a2, a3], axis=0)
    contrib = jnp.dot(astack, xl_ref[...], preferred_element_type=jnp.float32)
    ss_ref[...] = jnp.where(j == 0, 0.0, ss_ref[...]) + contrib

    @pl.when(j == pl.num_programs(0) - 1)
    def _():
        ss = ss_ref[...]
        s1, s2, s3 = ss[0:4], ss[4:8], ss[8:12]
        asum = jnp.sum(jnp.sum(asum_ref[...], axis=0), axis=1, keepdims=True)
        ad = asum[0:4]
        ap = asum[4:8]
        ehd = ehd_ref[...]
        cnt = ehd[:, 511:512]
        bgat = bgat_ref[...]
        ehad = jnp.concatenate([ehd[:, 0:511], ad], axis=1)
        numer = (s1 + NB * bgat
                 + jnp.dot(s2 + cnt * bgat, wmsg_ref[...],
                           preferred_element_type=jnp.float32)
                 + jnp.dot(s3 + cnt * bgat, wmsgi_ref[...],
                           preferred_element_type=jnp.float32)
                 + jnp.dot(ehad, wge_ref[...],
                           preferred_element_type=jnp.float32)
                 + cnt * (bmsg_ref[...] + bmsgi_ref[...] + bge_ref[...]))
        out_nodes = numer / NB
        out_edges = jnp.concatenate([ehp_ref[:, 0:511], ap], axis=1) / EB
        gg = _lrelu(jnp.dot(y2_ref[...], wgp_ref[...],
                            preferred_element_type=jnp.float32) + bgp_ref[...], 0.01)
        pooled = jnp.concatenate([out_nodes, out_edges, gg], axis=1)
        o = _lrelu(jnp.dot(pooled, wf1_ref[...],
                           preferred_element_type=jnp.float32) + bf1_ref[...], 0.01)
        out_ref[...] = jax.nn.sigmoid(
            jnp.dot(o, wf2_ref[...], preferred_element_type=jnp.float32)
            + bf2_ref[...])


def _k6(a1_p, a2_p, a3_p, x_l, ehd, ehp, asum_p, y2, W_gp, b_gp, b_gat,
        W_msg, b_msg, W_msg_i, b_msg_i, W_ge, b_ge, W_f1, b_f1, W_f2, b_f2):
    BN = 2048
    nj = NPAD // BN
    full = lambda shape: pl.BlockSpec(shape, lambda j: tuple(0 for _ in shape))
    outs = pl.pallas_call(
        _k6_body,
        grid=(nj,),
        in_specs=[
            pl.BlockSpec((NW, B, BN), lambda j: (0, 0, j)),
            pl.BlockSpec((NW, B, BN), lambda j: (0, 0, j)),
            pl.BlockSpec((NW, B, BN), lambda j: (0, 0, j)),
            pl.BlockSpec((BN, 1024), lambda j: (j, 0)),
            full((B, 512)), full((B, 512)), full((NW, 8, 16)),
            full((B, 5)), full((5, 512)), full((1, 512)),
            full((1, 1024)), full((1024, 1024)), full((1, 1024)),
            full((1024, 1024)), full((1, 1024)),
            full((512, 1024)), full((1, 1024)),
            full((2048, 256)), full((1, 256)), full((256, 1)), full((1, 1)),
        ],
        out_specs=[
            full((12, 1024)),
            full((B, 1)),
        ],
        out_shape=[
            jax.ShapeDtypeStruct((12, 1024), jnp.float32),
            jax.ShapeDtypeStruct((B, 1), jnp.float32),
        ],
        compiler_params=pltpu.CompilerParams(
            dimension_semantics=("arbitrary",)),
    )(a1_p, a2_p, a3_p, x_l, ehd, ehp, asum_p, y2, W_gp, b_gp, b_gat,
      W_msg, b_msg, W_msg_i, b_msg_i, W_ge, b_ge, W_f1, b_f1, W_f2, b_f2)
    return outs[1]


def kernel(x, edge_index, edge_attr, y, batch, W_node, b_node, W_ep, b_ep,
           W_gp, b_gp, W_l, b_l, W_r, b_r, W_e, att, b_gat, W_msg, b_msg,
           W_msg_i, b_msg_i, W_ge, b_ge, W_f1, b_f1, W_f2, b_f2):
    f32 = jnp.float32
    loop = jnp.arange(N, dtype=edge_index.dtype)
    pad1 = jnp.zeros((EPAD - ETOT,), jnp.int32)
    src = jnp.concatenate([edge_index[0], loop, pad1])
    dst = jnp.concatenate([edge_index[1], loop, pad1])
    ea_pad = jnp.concatenate(
        [edge_attr, jnp.ones((N, 5), f32), jnp.zeros((EPAD - ETOT, 5), f32)], axis=0)
    ec = jnp.minimum(jnp.arange(EPAD, dtype=jnp.int32) // EB, B - 1)
    dst3 = dst.reshape(G, 1, BLK)
    ec3 = ec.reshape(G, 1, BLK)
    x2 = jnp.concatenate([x[:, None], jnp.zeros((NPAD - N, 1), f32)], axis=0)
    y2 = y.reshape(B, 5)

    r2 = lambda a: a.reshape(1, -1)

    bf = jnp.bfloat16
    x_l, xl_p, xr_p = _k1(x2, r2(W_node), r2(b_node), W_l.astype(bf),
                          r2(b_l), W_r.astype(bf), r2(b_r))
    gl_i, gr_i = _g1(xl_p, xr_p, src, dst)
    lg3, ehd, ehp, mxv = _k2(ea_pad, gl_i, gr_i, dst3, ec3,
                             W_ep, r2(b_ep), W_e.astype(jnp.bfloat16), r2(att))
    lg = lg3.reshape(EPAD)
    mx16 = mxv[0, :16]
    ex, den_p, deg_p, c_p = _k3(lg, src, dst, mx16)
    den1, deg1, c1 = _k4(den_p, deg_p, c_p)
    al, a1_p, a3_p, asum_p = _k5a(ex, src, dst, den1.reshape(NPAD),
                                  deg1.reshape(NPAD))
    a2_p = _k5b(al, src, dst, c1.reshape(4 * NPAD))
    out = _k6(a1_p.reshape(NW, B, NPAD), a2_p.reshape(NW, B, NPAD),
              a3_p.reshape(NW, B, NPAD), x_l, ehd, ehp,
              asum_p.reshape(NW, 8, 16), y2,
              W_gp, r2(b_gp), r2(b_gat), W_msg, r2(b_msg), W_msg_i,
              r2(b_msg_i), W_ge, r2(b_ge), W_f1, r2(b_f1), W_f2, r2(b_f2))
    return out


# speedup vs baseline: 5.8111x; 1.0437x over previous
import functools

import jax
import jax.numpy as jnp
from jax import lax
from jax.experimental import pallas as pl
from jax.experimental.pallas import tpu as pltpu
from jax.experimental.pallas import tpu_sc as plsc

N = 10000
B = 4
NB = N // B
E = 160000
ETOT = E + N
EB = ETOT // B
BLK = 1024
G = (ETOT + BLK - 1) // BLK
EPAD = G * BLK
NPAD = 10240
NW = 32
EW = EPAD // NW
NEG = -1.0e30


def _lrelu(v, s):
    return jnp.where(v >= 0, v, s * v)


def _pack(v):
    bits = lambda a: lax.bitcast_convert_type(
        a.astype(jnp.bfloat16).astype(jnp.float32), jnp.int32)
    lo = bits(v[:, 0:512])
    hi = bits(v[:, 512:1024])
    return lax.shift_right_logical(lo, 16) | (hi & jnp.int32(-65536))


def _unpack(w):
    f_lo = lax.bitcast_convert_type(w << 16, jnp.float32)
    f_hi = lax.bitcast_convert_type(w & jnp.int32(-65536), jnp.float32)
    return f_lo, f_hi


def _k1_body(x_ref, wn_ref, bn_ref, wl_ref, bl_ref, wr_ref, br_ref,
             xl_ref, xlp_ref, xrp_ref):
    h = _lrelu(x_ref[...] * wn_ref[...] + bn_ref[...], 0.01).astype(jnp.bfloat16)
    xl = jnp.dot(h, wl_ref[...], preferred_element_type=jnp.float32) + bl_ref[...]
    xr = jnp.dot(h, wr_ref[...], preferred_element_type=jnp.float32) + br_ref[...]
    xl_ref[...] = xl
    xlp_ref[...] = _pack(xl)
    xrp_ref[...] = _pack(xr)


def _k1(x2, W_node, b_node, W_l, b_l, W_r, b_r):
    n_blk = NPAD // BLK
    full = lambda shape: pl.BlockSpec(shape, lambda g: (0, 0))
    return pl.pallas_call(
        _k1_body,
        grid=(n_blk,),
        in_specs=[
            pl.BlockSpec((BLK, 1), lambda g: (g, 0)),
            full((1, 512)), full((1, 512)),
            full((512, 1024)), full((1, 1024)),
            full((512, 1024)), full((1, 1024)),
        ],
        out_specs=[
            pl.BlockSpec((BLK, 1024), lambda g: (g, 0)),
            pl.BlockSpec((BLK, 512), lambda g: (g, 0)),
            pl.BlockSpec((BLK, 512), lambda g: (g, 0)),
        ],
        out_shape=[
            jax.ShapeDtypeStruct((NPAD, 1024), jnp.float32),
            jax.ShapeDtypeStruct((NPAD, 512), jnp.int32),
            jax.ShapeDtypeStruct((NPAD, 512), jnp.int32),
        ],
    )(x2, W_node, b_node, W_l, b_l, W_r, b_r)


GC = 48


def _g1(xl_i, xr_i, src_h, dst_h):
    n_edges = src_h.shape[0]
    EWH = n_edges // NW
    NCH = EWH // GC
    TAIL = EWH - NCH * GC
    GPP = (NCH - 1) // 2 * 2 if TAIL else NCH // 2 * 2

    def body(xl_hbm, xr_hbm, src_hbm, dst_hbm, gl_hbm, gr_hbm,
             src_v, dst_v, bufa, bufb, sga, sgb, swa0, swa1, swb0, swb1):
        wid = lax.axis_index("s") * 2 + lax.axis_index("c")
        base = pl.multiple_of(wid * EWH, 8)
        pltpu.sync_copy(src_hbm.at[pl.ds(base, EWH)], src_v)
        pltpu.sync_copy(dst_hbm.at[pl.ds(base, EWH)], dst_v)
        swa = (swa0, swa1)
        swb = (swb0, swb1)

        def gather(k, p, rows):
            da = pltpu.async_copy(xl_hbm.at[src_v.at[pl.ds(k * GC, rows)]],
                                  bufa.at[p].at[pl.ds(0, rows)], sga)
            db = pltpu.async_copy(xr_hbm.at[dst_v.at[pl.ds(k * GC, rows)]],
                                  bufb.at[p].at[pl.ds(0, rows)], sgb)
            da.wait()
            db.wait()

        def wr(k, p, rows, wait):
            dsta = gl_hbm.at[pl.ds(base + k * GC, rows)]
            dstb = gr_hbm.at[pl.ds(base + k * GC, rows)]
            if wait:
                pltpu.sync_copy(bufa.at[p].at[pl.ds(0, rows)], dsta)
                pltpu.sync_copy(bufb.at[p].at[pl.ds(0, rows)], dstb)
            else:
                pltpu.async_copy(bufa.at[p].at[pl.ds(0, rows)], dsta, swa[p])
                pltpu.async_copy(bufb.at[p].at[pl.ds(0, rows)], dstb, swb[p])

        def drain(p):
            pltpu.make_async_copy(bufa.at[p], gl_hbm.at[pl.ds(base, GC)],
                                  swa[p]).wait()
            pltpu.make_async_copy(bufb.at[p], gr_hbm.at[pl.ds(base, GC)],
                                  swb[p]).wait()

        def loop(o, _):
            for p in range(2):
                k = o * 2 + p

                @pl.when(o > 0)
                def _():
                    drain(p)

                gather(k, p, GC)
                wr(k, p, GC, wait=False)
            return _

        lax.fori_loop(0, GPP // 2, loop, None)
        drain(0)
        drain(1)
        for k in range(GPP, NCH):
            gather(k, 0, GC)
            wr(k, 0, GC, wait=True)
        if TAIL:
            gather(NCH, 0, TAIL)
            wr(NCH, 0, TAIL, wait=True)

    mesh = plsc.VectorSubcoreMesh(core_axis_name="c", subcore_axis_name="s")
    f = pl.kernel(
        body,
        out_type=(
            jax.ShapeDtypeStruct((n_edges, 512), jnp.int32),
            jax.ShapeDtypeStruct((n_edges, 512), jnp.int32),
        ),
        mesh=mesh,
        compiler_params=pltpu.CompilerParams(needs_layout_passes=False),
        scratch_types=[
            pltpu.VMEM((EWH,), jnp.int32),
            pltpu.VMEM((EWH,), jnp.int32),
            pltpu.VMEM((2, GC, 512), jnp.int32),
            pltpu.VMEM((2, GC, 512), jnp.int32),
            pltpu.SemaphoreType.DMA,
            pltpu.SemaphoreType.DMA,
            pltpu.SemaphoreType.DMA,
            pltpu.SemaphoreType.DMA,
            pltpu.SemaphoreType.DMA,
            pltpu.SemaphoreType.DMA,
        ],
    )
    return f(xl_i, xr_i, src_h, dst_h)


def _k2_body(ea_ref, gl_ref, gr_ref, dst_ref, ec_ref, wep_ref, bep_ref,
             we_ref, att_ref, lg_ref, ehd_ref, ehp_ref, mx_ref, *, goff):
    g = pl.program_id(0)
    eh = _lrelu(jnp.dot(ea_ref[...], wep_ref[...],
                        preferred_element_type=jnp.float32) + bep_ref[...], 0.01)
    z = jnp.dot(eh.astype(jnp.bfloat16), we_ref[...],
                preferred_element_type=jnp.float32)
    gl_lo, gl_hi = _unpack(gl_ref[...])
    gr_lo, gr_hi = _unpack(gr_ref[...])
    v_lo = z[:, 0:512] + gl_lo + gr_lo
    v_hi = z[:, 512:1024] + gl_hi + gr_hi
    att = att_ref[...]
    lg = (jnp.sum(_lrelu(v_lo, 0.2) * att[:, 0:512], axis=1, keepdims=True)
          + jnp.sum(_lrelu(v_hi, 0.2) * att[:, 512:1024], axis=1,
                    keepdims=True))

    col = jax.lax.broadcasted_iota(jnp.int32, (BLK, 1), 0) + goff + g * BLK
    lgv = jnp.where(col < ETOT, lg, NEG)
    lg_ref[0] = lgv

    bm = jnp.max(lgv)
    prev = jnp.where(g == 0, NEG, mx_ref[...])
    mx_ref[...] = jnp.maximum(prev, jnp.full((1, 128), bm))

    row = jax.lax.broadcasted_iota(jnp.int32, (1, BLK), 1) + goff + g * BLK
    valid2 = row < ETOT
    lanes = jax.lax.broadcasted_iota(jnp.int32, (B, BLK), 0)
    dstc = dst_ref[0] // NB
    oh_d = jnp.where((dstc == lanes) & valid2, 1.0, 0.0)
    oh_p = jnp.where((ec_ref[0] == lanes) & valid2, 1.0, 0.0)
    eh_aug = jnp.concatenate([eh, jnp.ones((BLK, 1), jnp.float32)], axis=1)

    d_contrib = jnp.dot(oh_d, eh_aug, preferred_element_type=jnp.float32)
    p_contrib = jnp.dot(oh_p, eh_aug, preferred_element_type=jnp.float32)
    ehd_ref[...] = jnp.where(g == 0, 0.0, ehd_ref[...]) + d_contrib
    ehp_ref[...] = jnp.where(g == 0, 0.0, ehp_ref[...]) + p_contrib


def _k2(ea_pad, gl, gr, dst3, ec3, W_ep, b_ep, W_e, att, goff):
    gh = dst3.shape[0]
    full = lambda shape: pl.BlockSpec(shape, lambda g: (0, 0))
    return pl.pallas_call(
        functools.partial(_k2_body, goff=goff),
        grid=(gh,),
        in_specs=[
            pl.BlockSpec((BLK, 5), lambda g: (g, 0)),
            pl.BlockSpec((BLK, 512), lambda g: (g, 0)),
            pl.BlockSpec((BLK, 512), lambda g: (g, 0)),

            pl.BlockSpec((1, 1, BLK), lambda g: (g, 0, 0)),
            pl.BlockSpec((1, 1, BLK), lambda g: (g, 0, 0)),
            full((5, 511)), full((1, 511)), full((511, 1024)), full((1, 1024)),
        ],
        out_specs=[
            pl.BlockSpec((1, BLK, 1), lambda g: (g, 0, 0)),
            full((B, 512)), full((B, 512)),
            pl.BlockSpec((1, 128), lambda g: (0, 0)),
        ],
        out_shape=[
            jax.ShapeDtypeStruct((gh, BLK, 1), jnp.float32),
            jax.ShapeDtypeStruct((B, 512), jnp.float32),
            jax.ShapeDtypeStruct((B, 512), jnp.float32),
            jax.ShapeDtypeStruct((1, 128), jnp.float32),
        ],
        compiler_params=pltpu.CompilerParams(
            dimension_semantics=("arbitrary",)),
    )(ea_pad, gl, gr, dst3, ec3, W_ep, b_ep, W_e, att)


def _k3_body(lg_hbm, src_hbm, dst_hbm, mx_hbm,
             ex_hbm, den_hbm, deg_hbm, c_hbm,
             lgv, srcv, dstv, mxv, denv, degv, cv):
    wid = lax.axis_index("s") * 2 + lax.axis_index("c")
    base = pl.multiple_of(wid * EW, 8)
    pltpu.sync_copy(lg_hbm.at[pl.ds(base, EW)], lgv)
    pltpu.sync_copy(src_hbm.at[pl.ds(base, EW)], srcv)
    pltpu.sync_copy(dst_hbm.at[pl.ds(base, EW)], dstv)
    pltpu.sync_copy(mx_hbm, mxv)

    def zero(i, ref):
        ref[pl.ds(i * 16, 16)] = jnp.zeros((16,), jnp.float32)

    lax.fori_loop(0, NPAD // 16, lambda i, _: (zero(i, denv), _)[1], None)
    lax.fori_loop(0, NPAD // 16, lambda i, _: (zero(i, degv), _)[1], None)
    lax.fori_loop(0, 4 * NPAD // 16, lambda i, _: (zero(i, cv), _)[1], None)

    mx16 = jnp.maximum(mxv[0], mxv[1])
    lane = jnp.arange(16, dtype=jnp.int32)
    ones = jnp.ones((16,), jnp.float32)

    def body(i, _):
        sl = pl.ds(i * 16, 16)
        ex16 = jnp.exp(lgv[sl] - mx16)
        lgv[sl] = ex16
        d16 = dstv[sl]
        s16 = srcv[sl]
        valid = (base + i * 16 + lane) < ETOT
        plsc.addupdate_scatter(denv, [d16], ex16)
        plsc.addupdate_scatter(degv, [d16], ones, mask=valid)
        m16 = d16 // NB
        plsc.addupdate_scatter(cv, [m16 * NPAD + s16], ones, mask=valid)
        return _

    lax.fori_loop(0, EW // 16, body, None)

    pltpu.sync_copy(lgv, ex_hbm.at[pl.ds(base, EW)])
    pltpu.sync_copy(denv, den_hbm.at[wid])
    pltpu.sync_copy(degv, deg_hbm.at[wid])
    pltpu.sync_copy(cv, c_hbm.at[wid])


def _k3(lg, src, dst, mx2):
    mesh = plsc.VectorSubcoreMesh(core_axis_name="c", subcore_axis_name="s")
    f = pl.kernel(
        _k3_body,
        out_type=(
            jax.ShapeDtypeStruct((EPAD,), jnp.float32),
            jax.ShapeDtypeStruct((NW, NPAD), jnp.float32),
            jax.ShapeDtypeStruct((NW, NPAD), jnp.float32),
            jax.ShapeDtypeStruct((NW, 4 * NPAD), jnp.float32),
        ),
        mesh=mesh,
        compiler_params=pltpu.CompilerParams(needs_layout_passes=False),
        scratch_types=[
            pltpu.VMEM((EW,), jnp.float32),
            pltpu.VMEM((EW,), jnp.int32),
            pltpu.VMEM((EW,), jnp.int32),
            pltpu.VMEM((2, 16), jnp.float32),
            pltpu.VMEM((NPAD,), jnp.float32),
            pltpu.VMEM((NPAD,), jnp.float32),
            pltpu.VMEM((4 * NPAD,), jnp.float32),
        ],
    )
    return f(lg, src, dst, mx2)


def _k4_body(denp_ref, degp_ref, cp_ref, den_ref, deg_ref, c_ref):
    den_ref[...] = jnp.sum(denp_ref[...], axis=0, keepdims=True)
    deg_ref[...] = jnp.sum(degp_ref[...], axis=0, keepdims=True)
    c_ref[...] = jnp.sum(cp_ref[...], axis=0, keepdims=True)


def _k4(den_p, deg_p, c_p):
    return pl.pallas_call(
        _k4_body,
        out_shape=[
            jax.ShapeDtypeStruct((1, NPAD), jnp.float32),
            jax.ShapeDtypeStruct((1, NPAD), jnp.float32),
            jax.ShapeDtypeStruct((1, 4 * NPAD), jnp.float32),
        ],
    )(den_p, deg_p, c_p)


def _k5a_body(ex_hbm, src_hbm, dst_hbm, den_hbm, deg_hbm,
              al_hbm, a1_hbm, a3_hbm, asum_hbm,
              exv, srcv, dstv, denv, degv, a1v, a3v, asumv):
    wid = lax.axis_index("s") * 2 + lax.axis_index("c")
    base = pl.multiple_of(wid * EW, 8)
    pltpu.sync_copy(ex_hbm.at[pl.ds(base, EW)], exv)
    pltpu.sync_copy(src_hbm.at[pl.ds(base, EW)], srcv)
    pltpu.sync_copy(dst_hbm.at[pl.ds(base, EW)], dstv)
    pltpu.sync_copy(den_hbm, denv)
    pltpu.sync_copy(deg_hbm, degv)

    def zero(i, ref):
        ref[pl.ds(i * 16, 16)] = jnp.zeros((16,), jnp.float32)

    lax.fori_loop(0, 4 * NPAD // 16, lambda i, _: (zero(i, a1v), _)[1], None)
    lax.fori_loop(0, 4 * NPAD // 16, lambda i, _: (zero(i, a3v), _)[1], None)
    for b in range(8):
        asumv[pl.ds(b * 16, 16)] = jnp.zeros((16,), jnp.float32)

    lane = jnp.arange(16, dtype=jnp.int32)
    zf = jnp.zeros((16,), jnp.float32)

    def body(i, _):
        sl = pl.ds(i * 16, 16)
        d16 = dstv[sl]
        s16 = srcv[sl]
        den16 = plsc.load_gather(denv, [d16])
        a16 = exv[sl] / den16
        exv[sl] = a16
        deg16 = plsc.load_gather(degv, [d16])
        m16 = d16 // NB
        fl = m16 * NPAD + s16
        plsc.addupdate_scatter(a1v, [fl], a16)
        plsc.addupdate_scatter(a3v, [fl], deg16 * a16)
        p16 = (base + i * 16 + lane) // EB
        for b in range(B):
            plsc.addupdate(asumv.at[pl.ds(b * 16, 16)],
                           jnp.where(m16 == b, a16, zf))
            plsc.addupdate(asumv.at[pl.ds((4 + b) * 16, 16)],
                           jnp.where(p16 == b, a16, zf))
        return _

    lax.fori_loop(0, EW // 16, body, None)

    pltpu.sync_copy(exv, al_hbm.at[pl.ds(base, EW)])
    pltpu.sync_copy(a1v, a1_hbm.at[wid])
    pltpu.sync_copy(a3v, a3_hbm.at[wid])
    pltpu.sync_copy(asumv, asum_hbm.at[wid])


def _k5a(ex, src, dst, den1, deg1):
    mesh = plsc.VectorSubcoreMesh(core_axis_name="c", subcore_axis_name="s")
    f = pl.kernel(
        _k5a_body,
        out_type=(
            jax.ShapeDtypeStruct((EPAD,), jnp.float32),
            jax.ShapeDtypeStruct((NW, 4 * NPAD), jnp.float32),
            jax.ShapeDtypeStruct((NW, 4 * NPAD), jnp.float32),
            jax.ShapeDtypeStruct((NW, 128), jnp.float32),
        ),
        mesh=mesh,
        compiler_params=pltpu.CompilerParams(needs_layout_passes=False),
        scratch_types=[
            pltpu.VMEM((EW,), jnp.float32),
            pltpu.VMEM((EW,), jnp.int32),
            pltpu.VMEM((EW,), jnp.int32),
            pltpu.VMEM((NPAD,), jnp.float32),
            pltpu.VMEM((NPAD,), jnp.float32),
            pltpu.VMEM((4 * NPAD,), jnp.float32),
            pltpu.VMEM((4 * NPAD,), jnp.float32),
            pltpu.VMEM((128,), jnp.float32),
        ],
    )
    return f(ex, src, dst, den1, deg1)


def _k5b_body(al_hbm, src_hbm, dst_hbm, c_hbm, a2_hbm,
              alv, srcv, dstv, cv, a2v):
    wid = lax.axis_index("s") * 2 + lax.axis_index("c")
    base = pl.multiple_of(wid * EW, 8)
    pltpu.sync_copy(al_hbm.at[pl.ds(base, EW)], alv)
    pltpu.sync_copy(src_hbm.at[pl.ds(base, EW)], srcv)
    pltpu.sync_copy(dst_hbm.at[pl.ds(base, EW)], dstv)
    pltpu.sync_copy(c_hbm, cv)

    def zero(i, ref):
        ref[pl.ds(i * 16, 16)] = jnp.zeros((16,), jnp.float32)

    lax.fori_loop(0, 4 * NPAD // 16, lambda i, _: (zero(i, a2v), _)[1], None)

    def body(i, _):
        sl = pl.ds(i * 16, 16)
        a16 = alv[sl]
        d16 = dstv[sl]
        s16 = srcv[sl]
        for b in range(B):
            cb = plsc.load_gather(cv, [b * NPAD + d16])
            plsc.addupdate_scatter(a2v, [b * NPAD + s16], cb * a16)
        return _

    lax.fori_loop(0, EW // 16, body, None)
    pltpu.sync_copy(a2v, a2_hbm.at[wid])


def _k5b(al, src, dst, c1):
    mesh = plsc.VectorSubcoreMesh(core_axis_name="c", subcore_axis_name="s")
    f = pl.kernel(
        _k5b_body,
        out_type=jax.ShapeDtypeStruct((NW, 4 * NPAD), jnp.float32),
        mesh=mesh,
        compiler_params=pltpu.CompilerParams(needs_layout_passes=False),
        scratch_types=[
            pltpu.VMEM((EW,), jnp.float32),
            pltpu.VMEM((EW,), jnp.int32),
            pltpu.VMEM((EW,), jnp.int32),
            pltpu.VMEM((4 * NPAD,), jnp.float32),
            pltpu.VMEM((4 * NPAD,), jnp.float32),
        ],
    )
    return f(al, src, dst, c1)


def _k6_body(a1_ref, a2_ref, a3_ref, xl_ref, ehd_ref, ehd2_ref, ehp_ref,
             ehp2_ref, asum_ref,
             y2_ref, wgp_ref, bgp_ref, bgat_ref, wmsg_ref, bmsg_ref,
             wmsgi_ref, bmsgi_ref, wge_ref, bge_ref, wf1_ref, bf1_ref,
             wf2_ref, bf2_ref, ss_ref, out_ref):
    j = pl.program_id(0)
    a1 = jnp.sum(a1_ref[...], axis=0)
    a2 = jnp.sum(a2_ref[...], axis=0)
    a3 = jnp.sum(a3_ref[...], axis=0)
    astack = jnp.concatenate([a1, a2, a3], axis=0)
    contrib = jnp.dot(astack, xl_ref[...], preferred_element_type=jnp.float32)
    ss_ref[...] = jnp.where(j == 0, 0.0, ss_ref[...]) + contrib

    @pl.when(j == pl.num_programs(0) - 1)
    def _():
        ss = ss_ref[...]
        s1, s2, s3 = ss[0:4], ss[4:8], ss[8:12]
        asum = jnp.sum(jnp.sum(asum_ref[...], axis=0), axis=1, keepdims=True)
        ad = asum[0:4]
        ap = asum[4:8]
        ehd = ehd_ref[...] + ehd2_ref[...]
        cnt = ehd[:, 511:512]
        bgat = bgat_ref[...]
        ehad = jnp.concatenate([ehd[:, 0:511], ad], axis=1)
        numer = (s1 + NB * bgat
                 + jnp.dot(s2 + cnt * bgat, wmsg_ref[...],
                           preferred_element_type=jnp.float32)
                 + jnp.dot(s3 + cnt * bgat, wmsgi_ref[...],
                           preferred_element_type=jnp.float32)
                 + jnp.dot(ehad, wge_ref[...],
                           preferred_element_type=jnp.float32)
                 + cnt * (bmsg_ref[...] + bmsgi_ref[...] + bge_ref[...]))
        out_nodes = numer / NB
        ehp = ehp_ref[...] + ehp2_ref[...]
        out_edges = jnp.concatenate([ehp[:, 0:511], ap], axis=1) / EB
        gg = _lrelu(jnp.dot(y2_ref[...], wgp_ref[...],
                            preferred_element_type=jnp.float32) + bgp_ref[...], 0.01)
        pooled = jnp.concatenate([out_nodes, out_edges, gg], axis=1)
        o = _lrelu(jnp.dot(pooled, wf1_ref[...],
                           preferred_element_type=jnp.float32) + bf1_ref[...], 0.01)
        out_ref[...] = jax.nn.sigmoid(
            jnp.dot(o, wf2_ref[...], preferred_element_type=jnp.float32)
            + bf2_ref[...])


def _k6(a1_p, a2_p, a3_p, x_l, ehd, ehd2, ehp, ehp2, asum_p, y2, W_gp,
        b_gp, b_gat, W_msg, b_msg, W_msg_i, b_msg_i, W_ge, b_ge, W_f1,
        b_f1, W_f2, b_f2):
    BN = 2048
    nj = NPAD // BN
    full = lambda shape: pl.BlockSpec(shape, lambda j: tuple(0 for _ in shape))
    outs = pl.pallas_call(
        _k6_body,
        grid=(nj,),
        in_specs=[
            pl.BlockSpec((NW, B, BN), lambda j: (0, 0, j)),
            pl.BlockSpec((NW, B, BN), lambda j: (0, 0, j)),
            pl.BlockSpec((NW, B, BN), lambda j: (0, 0, j)),
            pl.BlockSpec((BN, 1024), lambda j: (j, 0)),
            full((B, 512)), full((B, 512)), full((B, 512)), full((B, 512)),
            full((NW, 8, 16)),
            full((B, 5)), full((5, 512)), full((1, 512)),
            full((1, 1024)), full((1024, 1024)), full((1, 1024)),
            full((1024, 1024)), full((1, 1024)),
            full((512, 1024)), full((1, 1024)),
            full((2048, 256)), full((1, 256)), full((256, 1)), full((1, 1)),
        ],
        out_specs=[
            full((12, 1024)),
            full((B, 1)),
        ],
        out_shape=[
            jax.ShapeDtypeStruct((12, 1024), jnp.float32),
            jax.ShapeDtypeStruct((B, 1), jnp.float32),
        ],
        compiler_params=pltpu.CompilerParams(
            dimension_semantics=("arbitrary",)),
    )(a1_p, a2_p, a3_p, x_l, ehd, ehd2, ehp, ehp2, asum_p, y2, W_gp, b_gp,
      b_gat, W_msg, b_msg, W_msg_i, b_msg_i, W_ge, b_ge, W_f1, b_f1,
      W_f2, b_f2)
    return outs[1]


def kernel(x, edge_index, edge_attr, y, batch, W_node, b_node, W_ep, b_ep,
           W_gp, b_gp, W_l, b_l, W_r, b_r, W_e, att, b_gat, W_msg, b_msg,
           W_msg_i, b_msg_i, W_ge, b_ge, W_f1, b_f1, W_f2, b_f2):
    f32 = jnp.float32
    loop = jnp.arange(N, dtype=edge_index.dtype)
    pad1 = jnp.zeros((EPAD - ETOT,), jnp.int32)
    src = jnp.concatenate([edge_index[0], loop, pad1])
    dst = jnp.concatenate([edge_index[1], loop, pad1])
    ea_pad = jnp.concatenate(
        [edge_attr, jnp.ones((N, 5), f32), jnp.zeros((EPAD - ETOT, 5), f32)], axis=0)
    ec = jnp.minimum(jnp.arange(EPAD, dtype=jnp.int32) // EB, B - 1)
    dst3 = dst.reshape(G, 1, BLK)
    ec3 = ec.reshape(G, 1, BLK)
    x2 = jnp.concatenate([x[:, None], jnp.zeros((NPAD - N, 1), f32)], axis=0)
    y2 = y.reshape(B, 5)

    r2 = lambda a: a.reshape(1, -1)

    bf = jnp.bfloat16
    x_l, xl_p, xr_p = _k1(x2, r2(W_node), r2(b_node), W_l.astype(bf),
                          r2(b_l), W_r.astype(bf), r2(b_r))
    GH0 = 84
    EH0 = GH0 * BLK
    we_bf = W_e.astype(jnp.bfloat16)
    lgs, ehds, ehps, mxs = [], [], [], []
    for part, (lo, hi, goff) in enumerate([(0, EH0, 0), (EH0, EPAD, EH0)]):
        gl_i, gr_i = _g1(xl_p, xr_p, src[lo:hi], dst[lo:hi])
        lg3, ehd_h, ehp_h, mxv_h = _k2(
            ea_pad[lo:hi], gl_i, gr_i,
            dst3[lo // BLK:hi // BLK], ec3[lo // BLK:hi // BLK],
            W_ep, r2(b_ep), we_bf, r2(att), goff)
        lgs.append(lg3.reshape(hi - lo))
        ehds.append(ehd_h)
        ehps.append(ehp_h)
        mxs.append(mxv_h[0, :16])
    lg = jnp.concatenate(lgs)
    mx2 = jnp.stack(mxs)
    ex, den_p, deg_p, c_p = _k3(lg, src, dst, mx2)
    den1, deg1, c1 = _k4(den_p, deg_p, c_p)
    al, a1_p, a3_p, asum_p = _k5a(ex, src, dst, den1.reshape(NPAD),
                                  deg1.reshape(NPAD))
    a2_p = _k5b(al, src, dst, c1.reshape(4 * NPAD))
    out = _k6(a1_p.reshape(NW, B, NPAD), a2_p.reshape(NW, B, NPAD),
              a3_p.reshape(NW, B, NPAD), x_l, ehds[0], ehds[1], ehps[0],
              ehps[1], asum_p.reshape(NW, 8, 16), y2,
              W_gp, r2(b_gp), r2(b_gat), W_msg, r2(b_msg), W_msg_i,
              r2(b_msg_i), W_ge, r2(b_ge), W_f1, r2(b_f1), W_f2, r2(b_f2))
    return out


# speedup vs baseline: 5.8480x; 1.0063x over previous
import functools

import jax
import jax.numpy as jnp
from jax import lax
from jax.experimental import pallas as pl
from jax.experimental.pallas import tpu as pltpu
from jax.experimental.pallas import tpu_sc as plsc

N = 10000
B = 4
NB = N // B
E = 160000
ETOT = E + N
EB = ETOT // B
BLK = 1024
G = (ETOT + BLK - 1) // BLK
EPAD = G * BLK
NPAD = 10240
NW = 32
EW = EPAD // NW
NEG = -1.0e30


def _lrelu(v, s):
    return jnp.where(v >= 0, v, s * v)


def _pack(v):
    bits = lambda a: lax.bitcast_convert_type(
        a.astype(jnp.bfloat16).astype(jnp.float32), jnp.int32)
    lo = bits(v[:, 0:512])
    hi = bits(v[:, 512:1024])
    return lax.shift_right_logical(lo, 16) | (hi & jnp.int32(-65536))


def _unpack(w):
    f_lo = lax.bitcast_convert_type(w << 16, jnp.float32)
    f_hi = lax.bitcast_convert_type(w & jnp.int32(-65536), jnp.float32)
    return f_lo, f_hi


def _k1_body(x_ref, wn_ref, bn_ref, wl_ref, bl_ref, wr_ref, br_ref,
             xl_ref, xlp_ref, xrp_ref):
    h = _lrelu(x_ref[...] * wn_ref[...] + bn_ref[...], 0.01).astype(jnp.bfloat16)
    xl = jnp.dot(h, wl_ref[...], preferred_element_type=jnp.float32) + bl_ref[...]
    xr = jnp.dot(h, wr_ref[...], preferred_element_type=jnp.float32) + br_ref[...]
    xl_ref[...] = xl
    xlp_ref[...] = _pack(xl)
    xrp_ref[...] = _pack(xr)


def _k1(x2, W_node, b_node, W_l, b_l, W_r, b_r):
    n_blk = NPAD // BLK
    full = lambda shape: pl.BlockSpec(shape, lambda g: (0, 0))
    return pl.pallas_call(
        _k1_body,
        grid=(n_blk,),
        in_specs=[
            pl.BlockSpec((BLK, 1), lambda g: (g, 0)),
            full((1, 512)), full((1, 512)),
            full((512, 1024)), full((1, 1024)),
            full((512, 1024)), full((1, 1024)),
        ],
        out_specs=[
            pl.BlockSpec((BLK, 1024), lambda g: (g, 0)),
            pl.BlockSpec((BLK, 512), lambda g: (g, 0)),
            pl.BlockSpec((BLK, 512), lambda g: (g, 0)),
        ],
        out_shape=[
            jax.ShapeDtypeStruct((NPAD, 1024), jnp.float32),
            jax.ShapeDtypeStruct((NPAD, 512), jnp.int32),
            jax.ShapeDtypeStruct((NPAD, 512), jnp.int32),
        ],
    )(x2, W_node, b_node, W_l, b_l, W_r, b_r)


GC = 32
NP_ = 3


def _g1(xl_i, xr_i, src_h, dst_h):
    n_edges = src_h.shape[0]
    EWH = n_edges // NW
    NCH = EWH // GC
    assert NCH * GC == EWH and NCH >= 6

    def body(xl_hbm, xr_hbm, src_hbm, dst_hbm, gl_hbm, gr_hbm,
             src_v, dst_v, bufa, bufb, *sems):
        sgas, sgbs, swas, swbs = sems[0:3], sems[3:6], sems[6:9], sems[9:12]
        wid = lax.axis_index("s") * 2 + lax.axis_index("c")
        base = pl.multiple_of(wid * EWH, 8)
        pltpu.sync_copy(src_hbm.at[pl.ds(base, EWH)], src_v)
        pltpu.sync_copy(dst_hbm.at[pl.ds(base, EWH)], dst_v)

        def gissue(k, p):
            pltpu.async_copy(xl_hbm.at[src_v.at[pl.ds(k * GC, GC)]],
                             bufa.at[p], sgas[p])
            pltpu.async_copy(xr_hbm.at[dst_v.at[pl.ds(k * GC, GC)]],
                             bufb.at[p], sgbs[p])

        def gwait(k, p):
            pltpu.make_async_copy(xl_hbm.at[src_v.at[pl.ds(k * GC, GC)]],
                                  bufa.at[p], sgas[p]).wait()
            pltpu.make_async_copy(xr_hbm.at[dst_v.at[pl.ds(k * GC, GC)]],
                                  bufb.at[p], sgbs[p]).wait()

        def wissue(k, p):
            pltpu.async_copy(bufa.at[p], gl_hbm.at[pl.ds(base + k * GC, GC)],
                             swas[p])
            pltpu.async_copy(bufb.at[p], gr_hbm.at[pl.ds(base + k * GC, GC)],
                             swbs[p])

        def wdrain(p):
            pltpu.make_async_copy(bufa.at[p], gl_hbm.at[pl.ds(base, GC)],
                                  swas[p]).wait()
            pltpu.make_async_copy(bufb.at[p], gr_hbm.at[pl.ds(base, GC)],
                                  swbs[p]).wait()

        gissue(0, 0)
        gissue(1, 1)

        def step(k, p, p2, do_drain):
            if isinstance(do_drain, bool):
                if do_drain:
                    wdrain(p)
            else:
                @pl.when(do_drain)
                def _():
                    wdrain(p)

            gissue(k, p)
            gwait(k - 2, p2)
            wissue(k - 2, p2)

        M = (NCH - 2) // NP_
        REM = (NCH - 2) - M * NP_

        def loop(o, _):
            for j in range(NP_):
                k = NP_ * o + 2 + j
                cond = True if j else (o > 0)
                step(k, (2 + j) % NP_, j % NP_, cond)
            return _

        lax.fori_loop(0, M, loop, None)
        for j in range(REM):
            k = M * NP_ + 2 + j
            step(k, k % NP_, (k - 2) % NP_, bool(k >= NP_))
        for k in (NCH - 2, NCH - 1):
            p = k % NP_
            gwait(k, p)
            wissue(k, p)
        for p in range(NP_):
            wdrain(p)

    mesh = plsc.VectorSubcoreMesh(core_axis_name="c", subcore_axis_name="s")
    f = pl.kernel(
        body,
        out_type=(
            jax.ShapeDtypeStruct((n_edges, 512), jnp.int32),
            jax.ShapeDtypeStruct((n_edges, 512), jnp.int32),
        ),
        mesh=mesh,
        compiler_params=pltpu.CompilerParams(needs_layout_passes=False),
        scratch_types=[
            pltpu.VMEM((EWH,), jnp.int32),
            pltpu.VMEM((EWH,), jnp.int32),
            pltpu.VMEM((NP_, GC, 512), jnp.int32),
            pltpu.VMEM((NP_, GC, 512), jnp.int32),
        ] + [pltpu.SemaphoreType.DMA] * 12,
    )
    return f(xl_i, xr_i, src_h, dst_h)


def _k2_body(ea_ref, gl_ref, gr_ref, dst_ref, ec_ref, wep_ref, bep_ref,
             we_ref, att_ref, lg_ref, ehd_ref, ehp_ref, mx_ref, *, goff):
    g = pl.program_id(0)
    eh = _lrelu(jnp.dot(ea_ref[...], wep_ref[...],
                        preferred_element_type=jnp.float32) + bep_ref[...], 0.01)
    z = jnp.dot(eh.astype(jnp.bfloat16), we_ref[...],
                preferred_element_type=jnp.float32)
    gl_lo, gl_hi = _unpack(gl_ref[...])
    gr_lo, gr_hi = _unpack(gr_ref[...])
    v_lo = z[:, 0:512] + gl_lo + gr_lo
    v_hi = z[:, 512:1024] + gl_hi + gr_hi
    att = att_ref[...]
    lg = (jnp.sum(_lrelu(v_lo, 0.2) * att[:, 0:512], axis=1, keepdims=True)
          + jnp.sum(_lrelu(v_hi, 0.2) * att[:, 512:1024], axis=1,
                    keepdims=True))

    col = jax.lax.broadcasted_iota(jnp.int32, (BLK, 1), 0) + goff + g * BLK
    lgv = jnp.where(col < ETOT, lg, NEG)
    lg_ref[0] = lgv

    bm = jnp.max(lgv)
    prev = jnp.where(g == 0, NEG, mx_ref[...])
    mx_ref[...] = jnp.maximum(prev, jnp.full((1, 128), bm))

    row = jax.lax.broadcasted_iota(jnp.int32, (1, BLK), 1) + goff + g * BLK
    valid2 = row < ETOT
    lanes = jax.lax.broadcasted_iota(jnp.int32, (B, BLK), 0)
    dstc = dst_ref[0] // NB
    oh_d = jnp.where((dstc == lanes) & valid2, 1.0, 0.0)
    oh_p = jnp.where((ec_ref[0] == lanes) & valid2, 1.0, 0.0)
    eh_aug = jnp.concatenate([eh, jnp.ones((BLK, 1), jnp.float32)], axis=1)

    d_contrib = jnp.dot(oh_d, eh_aug, preferred_element_type=jnp.float32)
    p_contrib = jnp.dot(oh_p, eh_aug, preferred_element_type=jnp.float32)
    ehd_ref[...] = jnp.where(g == 0, 0.0, ehd_ref[...]) + d_contrib
    ehp_ref[...] = jnp.where(g == 0, 0.0, ehp_ref[...]) + p_contrib


def _k2(ea_pad, gl, gr, dst3, ec3, W_ep, b_ep, W_e, att, goff):
    gh = dst3.shape[0]
    full = lambda shape: pl.BlockSpec(shape, lambda g: (0, 0))
    return pl.pallas_call(
        functools.partial(_k2_body, goff=goff),
        grid=(gh,),
        in_specs=[
            pl.BlockSpec((BLK, 5), lambda g: (g, 0)),
            pl.BlockSpec((BLK, 512), lambda g: (g, 0)),
            pl.BlockSpec((BLK, 512), lambda g: (g, 0)),

            pl.BlockSpec((1, 1, BLK), lambda g: (g, 0, 0)),
            pl.BlockSpec((1, 1, BLK), lambda g: (g, 0, 0)),
            full((5, 511)), full((1, 511)), full((511, 1024)), full((1, 1024)),
        ],
        out_specs=[
            pl.BlockSpec((1, BLK, 1), lambda g: (g, 0, 0)),
            full((B, 512)), full((B, 512)),
            pl.BlockSpec((1, 128), lambda g: (0, 0)),
        ],
        out_shape=[
            jax.ShapeDtypeStruct((gh, BLK, 1), jnp.float32),
            jax.ShapeDtypeStruct((B, 512), jnp.float32),
            jax.ShapeDtypeStruct((B, 512), jnp.float32),
            jax.ShapeDtypeStruct((1, 128), jnp.float32),
        ],
        compiler_params=pltpu.CompilerParams(
            dimension_semantics=("arbitrary",)),
    )(ea_pad, gl, gr, dst3, ec3, W_ep, b_ep, W_e, att)


def _k3_body(lg_hbm, src_hbm, dst_hbm, mx_hbm,
             ex_hbm, den_hbm, deg_hbm, c_hbm,
             lgv, srcv, dstv, mxv, denv, degv, cv):
    wid = lax.axis_index("s") * 2 + lax.axis_index("c")
    base = pl.multiple_of(wid * EW, 8)
    pltpu.sync_copy(lg_hbm.at[pl.ds(base, EW)], lgv)
    pltpu.sync_copy(src_hbm.at[pl.ds(base, EW)], srcv)
    pltpu.sync_copy(dst_hbm.at[pl.ds(base, EW)], dstv)
    pltpu.sync_copy(mx_hbm, mxv)

    def zero(i, ref):
        ref[pl.ds(i * 16, 16)] = jnp.zeros((16,), jnp.float32)

    lax.fori_loop(0, NPAD // 16, lambda i, _: (zero(i, denv), _)[1], None)
    lax.fori_loop(0, NPAD // 16, lambda i, _: (zero(i, degv), _)[1], None)
    lax.fori_loop(0, 4 * NPAD // 16, lambda i, _: (zero(i, cv), _)[1], None)

    mx16 = jnp.maximum(mxv[0], mxv[1])
    lane = jnp.arange(16, dtype=jnp.int32)
    ones = jnp.ones((16,), jnp.float32)

    def body(i, _):
        sl = pl.ds(i * 16, 16)
        ex16 = jnp.exp(lgv[sl] - mx16)
        lgv[sl] = ex16
        d16 = dstv[sl]
        s16 = srcv[sl]
        valid = (base + i * 16 + lane) < ETOT
        plsc.addupdate_scatter(denv, [d16], ex16)
        plsc.addupdate_scatter(degv, [d16], ones, mask=valid)
        m16 = d16 // NB
        plsc.addupdate_scatter(cv, [m16 * NPAD + s16], ones, mask=valid)
        return _

    lax.fori_loop(0, EW // 16, body, None)

    pltpu.sync_copy(lgv, ex_hbm.at[pl.ds(base, EW)])
    pltpu.sync_copy(denv, den_hbm.at[wid])
    pltpu.sync_copy(degv, deg_hbm.at[wid])
    pltpu.sync_copy(cv, c_hbm.at[wid])


def _k3(lg, src, dst, mx2):
    mesh = plsc.VectorSubcoreMesh(core_axis_name="c", subcore_axis_name="s")
    f = pl.kernel(
        _k3_body,
        out_type=(
            jax.ShapeDtypeStruct((EPAD,), jnp.float32),
            jax.ShapeDtypeStruct((NW, NPAD), jnp.float32),
            jax.ShapeDtypeStruct((NW, NPAD), jnp.float32),
            jax.ShapeDtypeStruct((NW, 4 * NPAD), jnp.float32),
        ),
        mesh=mesh,
        compiler_params=pltpu.CompilerParams(needs_layout_passes=False),
        scratch_types=[
            pltpu.VMEM((EW,), jnp.float32),
            pltpu.VMEM((EW,), jnp.int32),
            pltpu.VMEM((EW,), jnp.int32),
            pltpu.VMEM((2, 16), jnp.float32),
            pltpu.VMEM((NPAD,), jnp.float32),
            pltpu.VMEM((NPAD,), jnp.float32),
            pltpu.VMEM((4 * NPAD,), jnp.float32),
        ],
    )
    return f(lg, src, dst, mx2)


def _k4_body(denp_ref, degp_ref, cp_ref, den_ref, deg_ref, c_ref):
    den_ref[...] = jnp.sum(denp_ref[...], axis=0, keepdims=True)
    deg_ref[...] = jnp.sum(degp_ref[...], axis=0, keepdims=True)
    c_ref[...] = jnp.sum(cp_ref[...], axis=0, keepdims=True)


def _k4(den_p, deg_p, c_p):
    return pl.pallas_call(
        _k4_body,
        out_shape=[
            jax.ShapeDtypeStruct((1, NPAD), jnp.float32),
            jax.ShapeDtypeStruct((1, NPAD), jnp.float32),
            jax.ShapeDtypeStruct((1, 4 * NPAD), jnp.float32),
        ],
    )(den_p, deg_p, c_p)


def _k5a_body(ex_hbm, src_hbm, dst_hbm, den_hbm, deg_hbm,
              al_hbm, a1_hbm, a3_hbm, asum_hbm,
              exv, srcv, dstv, denv, degv, a1v, a3v, asumv):
    wid = lax.axis_index("s") * 2 + lax.axis_index("c")
    base = pl.multiple_of(wid * EW, 8)
    pltpu.sync_copy(ex_hbm.at[pl.ds(base, EW)], exv)
    pltpu.sync_copy(src_hbm.at[pl.ds(base, EW)], srcv)
    pltpu.sync_copy(dst_hbm.at[pl.ds(base, EW)], dstv)
    pltpu.sync_copy(den_hbm, denv)
    pltpu.sync_copy(deg_hbm, degv)

    def zero(i, ref):
        ref[pl.ds(i * 16, 16)] = jnp.zeros((16,), jnp.float32)

    lax.fori_loop(0, 4 * NPAD // 16, lambda i, _: (zero(i, a1v), _)[1], None)
    lax.fori_loop(0, 4 * NPAD // 16, lambda i, _: (zero(i, a3v), _)[1], None)
    for b in range(8):
        asumv[pl.ds(b * 16, 16)] = jnp.zeros((16,), jnp.float32)

    lane = jnp.arange(16, dtype=jnp.int32)
    zf = jnp.zeros((16,), jnp.float32)

    def body(i, _):
        sl = pl.ds(i * 16, 16)
        d16 = dstv[sl]
        s16 = srcv[sl]
        den16 = plsc.load_gather(denv, [d16])
        a16 = exv[sl] / den16
        exv[sl] = a16
        deg16 = plsc.load_gather(degv, [d16])
        m16 = d16 // NB
        fl = m16 * NPAD + s16
        plsc.addupdate_scatter(a1v, [fl], a16)
        plsc.addupdate_scatter(a3v, [fl], deg16 * a16)
        p16 = (base + i * 16 + lane) // EB
        for b in range(B):
            plsc.addupdate(asumv.at[pl.ds(b * 16, 16)],
                           jnp.where(m16 == b, a16, zf))
            plsc.addupdate(asumv.at[pl.ds((4 + b) * 16, 16)],
                           jnp.where(p16 == b, a16, zf))
        return _

    lax.fori_loop(0, EW // 16, body, None)

    pltpu.sync_copy(exv, al_hbm.at[pl.ds(base, EW)])
    pltpu.sync_copy(a1v, a1_hbm.at[wid])
    pltpu.sync_copy(a3v, a3_hbm.at[wid])
    pltpu.sync_copy(asumv, asum_hbm.at[wid])


def _k5a(ex, src, dst, den1, deg1):
    mesh = plsc.VectorSubcoreMesh(core_axis_name="c", subcore_axis_name="s")
    f = pl.kernel(
        _k5a_body,
        out_type=(
            jax.ShapeDtypeStruct((EPAD,), jnp.float32),
            jax.ShapeDtypeStruct((NW, 4 * NPAD), jnp.float32),
            jax.ShapeDtypeStruct((NW, 4 * NPAD), jnp.float32),
            jax.ShapeDtypeStruct((NW, 128), jnp.float32),
        ),
        mesh=mesh,
        compiler_params=pltpu.CompilerParams(needs_layout_passes=False),
        scratch_types=[
            pltpu.VMEM((EW,), jnp.float32),
            pltpu.VMEM((EW,), jnp.int32),
            pltpu.VMEM((EW,), jnp.int32),
            pltpu.VMEM((NPAD,), jnp.float32),
            pltpu.VMEM((NPAD,), jnp.float32),
            pltpu.VMEM((4 * NPAD,), jnp.float32),
            pltpu.VMEM((4 * NPAD,), jnp.float32),
            pltpu.VMEM((128,), jnp.float32),
        ],
    )
    return f(ex, src, dst, den1, deg1)


def _k5b_body(al_hbm, src_hbm, dst_hbm, c_hbm, a2_hbm,
              alv, srcv, dstv, cv, a2v):
    wid = lax.axis_index("s") * 2 + lax.axis_index("c")
    base = pl.multiple_of(wid * EW, 8)
    pltpu.sync_copy(al_hbm.at[pl.ds(base, EW)], alv)
    pltpu.sync_copy(src_hbm.at[pl.ds(base, EW)], srcv)
    pltpu.sync_copy(dst_hbm.at[pl.ds(base, EW)], dstv)
    pltpu.sync_copy(c_hbm, cv)

    def zero(i, ref):
        ref[pl.ds(i * 16, 16)] = jnp.zeros((16,), jnp.float32)

    lax.fori_loop(0, 4 * NPAD // 16, lambda i, _: (zero(i, a2v), _)[1], None)

    def body(i, _):
        sl = pl.ds(i * 16, 16)
        a16 = alv[sl]
        d16 = dstv[sl]
        s16 = srcv[sl]
        for b in range(B):
            cb = plsc.load_gather(cv, [b * NPAD + d16])
            plsc.addupdate_scatter(a2v, [b * NPAD + s16], cb * a16)
        return _

    lax.fori_loop(0, EW // 16, body, None)
    pltpu.sync_copy(a2v, a2_hbm.at[wid])


def _k5b(al, src, dst, c1):
    mesh = plsc.VectorSubcoreMesh(core_axis_name="c", subcore_axis_name="s")
    f = pl.kernel(
        _k5b_body,
        out_type=jax.ShapeDtypeStruct((NW, 4 * NPAD), jnp.float32),
        mesh=mesh,
        compiler_params=pltpu.CompilerParams(needs_layout_passes=False),
        scratch_types=[
            pltpu.VMEM((EW,), jnp.float32),
            pltpu.VMEM((EW,), jnp.int32),
            pltpu.VMEM((EW,), jnp.int32),
            pltpu.VMEM((4 * NPAD,), jnp.float32),
            pltpu.VMEM((4 * NPAD,), jnp.float32),
        ],
    )
    return f(al, src, dst, c1)


def _k6_body(a1_ref, a2_ref, a3_ref, xl_ref, ehd_ref, ehd2_ref, ehp_ref,
             ehp2_ref, asum_ref,
             y2_ref, wgp_ref, bgp_ref, bgat_ref, wmsg_ref, bmsg_ref,
             wmsgi_ref, bmsgi_ref, wge_ref, bge_ref, wf1_ref, bf1_ref,
             wf2_ref, bf2_ref, ss_ref, out_ref):
    j = pl.program_id(0)
    a1 = jnp.sum(a1_ref[...], axis=0)
    a2 = jnp.sum(a2_ref[...], axis=0)
    a3 = jnp.sum(a3_ref[...], axis=0)
    astack = jnp.concatenate([a1, a2, a3], axis=0)
    contrib = jnp.dot(astack, xl_ref[...], preferred_element_type=jnp.float32)
    ss_ref[...] = jnp.where(j == 0, 0.0, ss_ref[...]) + contrib

    @pl.when(j == pl.num_programs(0) - 1)
    def _():
        ss = ss_ref[...]
        s1, s2, s3 = ss[0:4], ss[4:8], ss[8:12]
        asum = jnp.sum(jnp.sum(asum_ref[...], axis=0), axis=1, keepdims=True)
        ad = asum[0:4]
        ap = asum[4:8]
        ehd = ehd_ref[...] + ehd2_ref[...]
        cnt = ehd[:, 511:512]
        bgat = bgat_ref[...]
        ehad = jnp.concatenate([ehd[:, 0:511], ad], axis=1)
        numer = (s1 + NB * bgat
                 + jnp.dot(s2 + cnt * bgat, wmsg_ref[...],
                           preferred_element_type=jnp.float32)
                 + jnp.dot(s3 + cnt * bgat, wmsgi_ref[...],
                           preferred_element_type=jnp.float32)
                 + jnp.dot(ehad, wge_ref[...],
                           preferred_element_type=jnp.float32)
                 + cnt * (bmsg_ref[...] + bmsgi_ref[...] + bge_ref[...]))
        out_nodes = numer / NB
        ehp = ehp_ref[...] + ehp2_ref[...]
        out_edges = jnp.concatenate([ehp[:, 0:511], ap], axis=1) / EB
        gg = _lrelu(jnp.dot(y2_ref[...], wgp_ref[...],
                            preferred_element_type=jnp.float32) + bgp_ref[...], 0.01)
        pooled = jnp.concatenate([out_nodes, out_edges, gg], axis=1)
        o = _lrelu(jnp.dot(pooled, wf1_ref[...],
                           preferred_element_type=jnp.float32) + bf1_ref[...], 0.01)
        out_ref[...] = jax.nn.sigmoid(
            jnp.dot(o, wf2_ref[...], preferred_element_type=jnp.float32)
            + bf2_ref[...])


def _k6(a1_p, a2_p, a3_p, x_l, ehd, ehd2, ehp, ehp2, asum_p, y2, W_gp,
        b_gp, b_gat, W_msg, b_msg, W_msg_i, b_msg_i, W_ge, b_ge, W_f1,
        b_f1, W_f2, b_f2):
    BN = 2048
    nj = NPAD // BN
    full = lambda shape: pl.BlockSpec(shape, lambda j: tuple(0 for _ in shape))
    outs = pl.pallas_call(
        _k6_body,
        grid=(nj,),
        in_specs=[
            pl.BlockSpec((NW, B, BN), lambda j: (0, 0, j)),
            pl.BlockSpec((NW, B, BN), lambda j: (0, 0, j)),
            pl.BlockSpec((NW, B, BN), lambda j: (0, 0, j)),
            pl.BlockSpec((BN, 1024), lambda j: (j, 0)),
            full((B, 512)), full((B, 512)), full((B, 512)), full((B, 512)),
            full((NW, 8, 16)),
            full((B, 5)), full((5, 512)), full((1, 512)),
            full((1, 1024)), full((1024, 1024)), full((1, 1024)),
            full((1024, 1024)), full((1, 1024)),
            full((512, 1024)), full((1, 1024)),
            full((2048, 256)), full((1, 256)), full((256, 1)), full((1, 1)),
        ],
        out_specs=[
            full((12, 1024)),
            full((B, 1)),
        ],
        out_shape=[
            jax.ShapeDtypeStruct((12, 1024), jnp.float32),
            jax.ShapeDtypeStruct((B, 1), jnp.float32),
        ],
        compiler_params=pltpu.CompilerParams(
            dimension_semantics=("arbitrary",)),
    )(a1_p, a2_p, a3_p, x_l, ehd, ehd2, ehp, ehp2, asum_p, y2, W_gp, b_gp,
      b_gat, W_msg, b_msg, W_msg_i, b_msg_i, W_ge, b_ge, W_f1, b_f1,
      W_f2, b_f2)
    return outs[1]


def kernel(x, edge_index, edge_attr, y, batch, W_node, b_node, W_ep, b_ep,
           W_gp, b_gp, W_l, b_l, W_r, b_r, W_e, att, b_gat, W_msg, b_msg,
           W_msg_i, b_msg_i, W_ge, b_ge, W_f1, b_f1, W_f2, b_f2):
    f32 = jnp.float32
    loop = jnp.arange(N, dtype=edge_index.dtype)
    pad1 = jnp.zeros((EPAD - ETOT,), jnp.int32)
    src = jnp.concatenate([edge_index[0], loop, pad1])
    dst = jnp.concatenate([edge_index[1], loop, pad1])
    ea_pad = jnp.concatenate(
        [edge_attr, jnp.ones((N, 5), f32), jnp.zeros((EPAD - ETOT, 5), f32)], axis=0)
    ec = jnp.minimum(jnp.arange(EPAD, dtype=jnp.int32) // EB, B - 1)
    dst3 = dst.reshape(G, 1, BLK)
    ec3 = ec.reshape(G, 1, BLK)
    x2 = jnp.concatenate([x[:, None], jnp.zeros((NPAD - N, 1), f32)], axis=0)
    y2 = y.reshape(B, 5)

    r2 = lambda a: a.reshape(1, -1)

    bf = jnp.bfloat16
    x_l, xl_p, xr_p = _k1(x2, r2(W_node), r2(b_node), W_l.astype(bf),
                          r2(b_l), W_r.astype(bf), r2(b_r))
    GH0 = 84
    EH0 = GH0 * BLK
    we_bf = W_e.astype(jnp.bfloat16)
    lgs, ehds, ehps, mxs = [], [], [], []
    for part, (lo, hi, goff) in enumerate([(0, EH0, 0), (EH0, EPAD, EH0)]):
        gl_i, gr_i = _g1(xl_p, xr_p, src[lo:hi], dst[lo:hi])
        lg3, ehd_h, ehp_h, mxv_h = _k2(
            ea_pad[lo:hi], gl_i, gr_i,
            dst3[lo // BLK:hi // BLK], ec3[lo // BLK:hi // BLK],
            W_ep, r2(b_ep), we_bf, r2(att), goff)
        lgs.append(lg3.reshape(hi - lo))
        ehds.append(ehd_h)
        ehps.append(ehp_h)
        mxs.append(mxv_h[0, :16])
    lg = jnp.concatenate(lgs)
    mx2 = jnp.stack(mxs)
    ex, den_p, deg_p, c_p = _k3(lg, src, dst, mx2)
    den1, deg1, c1 = _k4(den_p, deg_p, c_p)
    al, a1_p, a3_p, asum_p = _k5a(ex, src, dst, den1.reshape(NPAD),
                                  deg1.reshape(NPAD))
    a2_p = _k5b(al, src, dst, c1.reshape(4 * NPAD))
    out = _k6(a1_p.reshape(NW, B, NPAD), a2_p.reshape(NW, B, NPAD),
              a3_p.reshape(NW, B, NPAD), x_l, ehds[0], ehds[1], ehps[0],
              ehps[1], asum_p.reshape(NW, 8, 16), y2,
              W_gp, r2(b_gp), r2(b_gat), W_msg, r2(b_msg), W_msg_i,
              r2(b_msg_i), W_ge, r2(b_ge), W_f1, r2(b_f1), W_f2, r2(b_f2))
    return out


# speedup vs baseline: 5.9493x; 1.0173x over previous
import functools

import jax
import jax.numpy as jnp
from jax import lax
from jax.experimental import pallas as pl
from jax.experimental.pallas import tpu as pltpu
from jax.experimental.pallas import tpu_sc as plsc

N = 10000
B = 4
NB = N // B
E = 160000
ETOT = E + N
EB = ETOT // B
BLK = 1024
G = (ETOT + BLK - 1) // BLK
EPAD = G * BLK
NPAD = 10240
NW = 32
EW = EPAD // NW
NEG = -1.0e30


def _lrelu(v, s):
    return jnp.where(v >= 0, v, s * v)


def _pack(v):
    bits = lambda a: lax.bitcast_convert_type(
        a.astype(jnp.bfloat16).astype(jnp.float32), jnp.int32)
    lo = bits(v[:, 0:512])
    hi = bits(v[:, 512:1024])
    return lax.shift_right_logical(lo, 16) | (hi & jnp.int32(-65536))


def _unpack(w):
    f_lo = lax.bitcast_convert_type(w << 16, jnp.float32)
    f_hi = lax.bitcast_convert_type(w & jnp.int32(-65536), jnp.float32)
    return f_lo, f_hi


def _k1_body(x_ref, wn_ref, bn_ref, wl_ref, bl_ref, wr_ref, br_ref,
             xl_ref, xlp_ref, xrp_ref):
    h = _lrelu(x_ref[...] * wn_ref[...] + bn_ref[...], 0.01).astype(jnp.bfloat16)
    xl = jnp.dot(h, wl_ref[...], preferred_element_type=jnp.float32) + bl_ref[...]
    xr = jnp.dot(h, wr_ref[...], preferred_element_type=jnp.float32) + br_ref[...]
    xl_ref[...] = xl
    xlp_ref[...] = _pack(xl)
    xrp_ref[...] = _pack(xr)


def _k1(x2, W_node, b_node, W_l, b_l, W_r, b_r):
    n_blk = NPAD // BLK
    full = lambda shape: pl.BlockSpec(shape, lambda g: (0, 0))
    return pl.pallas_call(
        _k1_body,
        grid=(n_blk,),
        in_specs=[
            pl.BlockSpec((BLK, 1), lambda g: (g, 0)),
            full((1, 512)), full((1, 512)),
            full((512, 1024)), full((1, 1024)),
            full((512, 1024)), full((1, 1024)),
        ],
        out_specs=[
            pl.BlockSpec((BLK, 1024), lambda g: (g, 0)),
            pl.BlockSpec((BLK, 512), lambda g: (g, 0)),
            pl.BlockSpec((BLK, 512), lambda g: (g, 0)),
        ],
        out_shape=[
            jax.ShapeDtypeStruct((NPAD, 1024), jnp.float32),
            jax.ShapeDtypeStruct((NPAD, 512), jnp.int32),
            jax.ShapeDtypeStruct((NPAD, 512), jnp.int32),
        ],
    )(x2, W_node, b_node, W_l, b_l, W_r, b_r)


GC = 32
NP_ = 3


def _g1(xl_i, xr_i, src_h, dst_h):
    n_edges = src_h.shape[0]
    EWH = n_edges // NW
    NCH = EWH // GC
    assert NCH * GC == EWH and NCH >= 6

    def body(xl_hbm, xr_hbm, src_hbm, dst_hbm, gl_hbm, gr_hbm,
             src_v, dst_v, bufa, bufb, *sems):
        sgas, sgbs, swas, swbs = sems[0:3], sems[3:6], sems[6:9], sems[9:12]
        wid = lax.axis_index("s") * 2 + lax.axis_index("c")
        base = pl.multiple_of(wid * EWH, 8)
        pltpu.sync_copy(src_hbm.at[pl.ds(base, EWH)], src_v)
        pltpu.sync_copy(dst_hbm.at[pl.ds(base, EWH)], dst_v)

        def gissue(k, p):
            pltpu.async_copy(xl_hbm.at[src_v.at[pl.ds(k * GC, GC)]],
                             bufa.at[p], sgas[p])
            pltpu.async_copy(xr_hbm.at[dst_v.at[pl.ds(k * GC, GC)]],
                             bufb.at[p], sgbs[p])

        def gwait(k, p):
            pltpu.make_async_copy(xl_hbm.at[src_v.at[pl.ds(k * GC, GC)]],
                                  bufa.at[p], sgas[p]).wait()
            pltpu.make_async_copy(xr_hbm.at[dst_v.at[pl.ds(k * GC, GC)]],
                                  bufb.at[p], sgbs[p]).wait()

        def wissue(k, p):
            pltpu.async_copy(bufa.at[p], gl_hbm.at[pl.ds(base + k * GC, GC)],
                             swas[p])
            pltpu.async_copy(bufb.at[p], gr_hbm.at[pl.ds(base + k * GC, GC)],
                             swbs[p])

        def wdrain(p):
            pltpu.make_async_copy(bufa.at[p], gl_hbm.at[pl.ds(base, GC)],
                                  swas[p]).wait()
            pltpu.make_async_copy(bufb.at[p], gr_hbm.at[pl.ds(base, GC)],
                                  swbs[p]).wait()

        gissue(0, 0)
        gissue(1, 1)

        def step(k, p, p2, do_drain):
            if isinstance(do_drain, bool):
                if do_drain:
                    wdrain(p)
            else:
                @pl.when(do_drain)
                def _():
                    wdrain(p)

            gissue(k, p)
            gwait(k - 2, p2)
            wissue(k - 2, p2)

        M = (NCH - 2) // NP_
        REM = (NCH - 2) - M * NP_

        def loop(o, _):
            for j in range(NP_):
                k = NP_ * o + 2 + j
                cond = True if j else (o > 0)
                step(k, (2 + j) % NP_, j % NP_, cond)
            return _

        lax.fori_loop(0, M, loop, None)
        for j in range(REM):
            k = M * NP_ + 2 + j
            step(k, k % NP_, (k - 2) % NP_, bool(k >= NP_))
        for k in (NCH - 2, NCH - 1):
            p = k % NP_
            gwait(k, p)
            wissue(k, p)
        for p in range(NP_):
            wdrain(p)

    mesh = plsc.VectorSubcoreMesh(core_axis_name="c", subcore_axis_name="s")
    f = pl.kernel(
        body,
        out_type=(
            jax.ShapeDtypeStruct((n_edges, 512), jnp.int32),
            jax.ShapeDtypeStruct((n_edges, 512), jnp.int32),
        ),
        mesh=mesh,
        compiler_params=pltpu.CompilerParams(needs_layout_passes=False),
        scratch_types=[
            pltpu.VMEM((EWH,), jnp.int32),
            pltpu.VMEM((EWH,), jnp.int32),
            pltpu.VMEM((NP_, GC, 512), jnp.int32),
            pltpu.VMEM((NP_, GC, 512), jnp.int32),
        ] + [pltpu.SemaphoreType.DMA] * 12,
    )
    return f(xl_i, xr_i, src_h, dst_h)


def _k2_body(ea_ref, gl_ref, gr_ref, dst_ref, ec_ref, wep_ref, bep_ref,
             we_ref, att_ref, lg_ref, ehd_ref, ehp_ref, mx_ref, *, goff):
    g = pl.program_id(0)
    eh = _lrelu(jnp.dot(ea_ref[...], wep_ref[...],
                        preferred_element_type=jnp.float32) + bep_ref[...], 0.01)
    z = jnp.dot(eh.astype(jnp.bfloat16), we_ref[...],
                preferred_element_type=jnp.float32)
    gl_lo, gl_hi = _unpack(gl_ref[...])
    gr_lo, gr_hi = _unpack(gr_ref[...])
    v_lo = z[:, 0:512] + gl_lo + gr_lo
    v_hi = z[:, 512:1024] + gl_hi + gr_hi
    att = att_ref[...]
    lg = (jnp.sum(_lrelu(v_lo, 0.2) * att[:, 0:512], axis=1, keepdims=True)
          + jnp.sum(_lrelu(v_hi, 0.2) * att[:, 512:1024], axis=1,
                    keepdims=True))

    col = jax.lax.broadcasted_iota(jnp.int32, (BLK, 1), 0) + goff + g * BLK
    lgv = jnp.where(col < ETOT, lg, NEG)
    lg_ref[0] = lgv

    bm = jnp.max(lgv)
    prev = jnp.where(g == 0, NEG, mx_ref[...])
    mx_ref[...] = jnp.maximum(prev, jnp.full((1, 128), bm))

    row = jax.lax.broadcasted_iota(jnp.int32, (1, BLK), 1) + goff + g * BLK
    valid2 = row < ETOT
    lanes = jax.lax.broadcasted_iota(jnp.int32, (B, BLK), 0)
    dstc = dst_ref[0] // NB
    oh_d = jnp.where((dstc == lanes) & valid2, 1.0, 0.0)
    oh_p = jnp.where((ec_ref[0] == lanes) & valid2, 1.0, 0.0)
    eh_aug = jnp.concatenate([eh, jnp.ones((BLK, 1), jnp.float32)], axis=1)

    d_contrib = jnp.dot(oh_d, eh_aug, preferred_element_type=jnp.float32)
    p_contrib = jnp.dot(oh_p, eh_aug, preferred_element_type=jnp.float32)
    ehd_ref[...] = jnp.where(g == 0, 0.0, ehd_ref[...]) + d_contrib
    ehp_ref[...] = jnp.where(g == 0, 0.0, ehp_ref[...]) + p_contrib


def _k2(ea_pad, gl, gr, dst3, ec3, W_ep, b_ep, W_e, att, goff):
    gh = dst3.shape[0]
    full = lambda shape: pl.BlockSpec(shape, lambda g: (0, 0))
    return pl.pallas_call(
        functools.partial(_k2_body, goff=goff),
        grid=(gh,),
        in_specs=[
            pl.BlockSpec((BLK, 5), lambda g: (g, 0)),
            pl.BlockSpec((BLK, 512), lambda g: (g, 0)),
            pl.BlockSpec((BLK, 512), lambda g: (g, 0)),

            pl.BlockSpec((1, 1, BLK), lambda g: (g, 0, 0)),
            pl.BlockSpec((1, 1, BLK), lambda g: (g, 0, 0)),
            full((5, 511)), full((1, 511)), full((511, 1024)), full((1, 1024)),
        ],
        out_specs=[
            pl.BlockSpec((1, BLK, 1), lambda g: (g, 0, 0)),
            full((B, 512)), full((B, 512)),
            pl.BlockSpec((1, 128), lambda g: (0, 0)),
        ],
        out_shape=[
            jax.ShapeDtypeStruct((gh, BLK, 1), jnp.float32),
            jax.ShapeDtypeStruct((B, 512), jnp.float32),
            jax.ShapeDtypeStruct((B, 512), jnp.float32),
            jax.ShapeDtypeStruct((1, 128), jnp.float32),
        ],
        compiler_params=pltpu.CompilerParams(
            dimension_semantics=("arbitrary",)),
    )(ea_pad, gl, gr, dst3, ec3, W_ep, b_ep, W_e, att)


def _k3_body(lg_hbm, src_hbm, dst_hbm, mx_hbm,
             ex_hbm, den_hbm, deg_hbm, c_hbm,
             lgv, srcv, dstv, mxv, denv, degv, cv):
    wid = lax.axis_index("s") * 2 + lax.axis_index("c")
    base = pl.multiple_of(wid * EW, 8)
    pltpu.sync_copy(lg_hbm.at[pl.ds(base, EW)], lgv)
    pltpu.sync_copy(src_hbm.at[pl.ds(base, EW)], srcv)
    pltpu.sync_copy(dst_hbm.at[pl.ds(base, EW)], dstv)
    pltpu.sync_copy(mx_hbm, mxv)

    def zero(i, ref):
        ref[pl.ds(i * 16, 16)] = jnp.zeros((16,), jnp.float32)

    lax.fori_loop(0, NPAD // 16, lambda i, _: (zero(i, denv), _)[1], None)
    lax.fori_loop(0, NPAD // 16, lambda i, _: (zero(i, degv), _)[1], None)
    lax.fori_loop(0, 4 * NPAD // 16, lambda i, _: (zero(i, cv), _)[1], None)

    mx16 = jnp.maximum(jnp.maximum(mxv[0], mxv[1]),
                       jnp.maximum(mxv[2], mxv[3]))
    lane = jnp.arange(16, dtype=jnp.int32)
    ones = jnp.ones((16,), jnp.float32)

    def body(i, _):
        sl = pl.ds(i * 16, 16)
        ex16 = jnp.exp(lgv[sl] - mx16)
        lgv[sl] = ex16
        d16 = dstv[sl]
        s16 = srcv[sl]
        valid = (base + i * 16 + lane) < ETOT
        plsc.addupdate_scatter(denv, [d16], ex16)
        plsc.addupdate_scatter(degv, [d16], ones, mask=valid)
        m16 = d16 // NB
        plsc.addupdate_scatter(cv, [m16 * NPAD + s16], ones, mask=valid)
        return _

    lax.fori_loop(0, EW // 16, body, None)

    pltpu.sync_copy(lgv, ex_hbm.at[pl.ds(base, EW)])
    pltpu.sync_copy(denv, den_hbm.at[wid])
    pltpu.sync_copy(degv, deg_hbm.at[wid])
    pltpu.sync_copy(cv, c_hbm.at[wid])


def _k3(lg, src, dst, mx2):
    mesh = plsc.VectorSubcoreMesh(core_axis_name="c", subcore_axis_name="s")
    f = pl.kernel(
        _k3_body,
        out_type=(
            jax.ShapeDtypeStruct((EPAD,), jnp.float32),
            jax.ShapeDtypeStruct((NW, NPAD), jnp.float32),
            jax.ShapeDtypeStruct((NW, NPAD), jnp.float32),
            jax.ShapeDtypeStruct((NW, 4 * NPAD), jnp.float32),
        ),
        mesh=mesh,
        compiler_params=pltpu.CompilerParams(needs_layout_passes=False),
        scratch_types=[
            pltpu.VMEM((EW,), jnp.float32),
            pltpu.VMEM((EW,), jnp.int32),
            pltpu.VMEM((EW,), jnp.int32),
            pltpu.VMEM((4, 16), jnp.float32),
            pltpu.VMEM((NPAD,), jnp.float32),
            pltpu.VMEM((NPAD,), jnp.float32),
            pltpu.VMEM((4 * NPAD,), jnp.float32),
        ],
    )
    return f(lg, src, dst, mx2)


def _k4_body(denp_ref, degp_ref, cp_ref, den_ref, deg_ref, c_ref):
    den_ref[...] = jnp.sum(denp_ref[...], axis=0, keepdims=True)
    deg_ref[...] = jnp.sum(degp_ref[...], axis=0, keepdims=True)
    c_ref[...] = jnp.sum(cp_ref[...], axis=0, keepdims=True)


def _k4(den_p, deg_p, c_p):
    return pl.pallas_call(
        _k4_body,
        out_shape=[
            jax.ShapeDtypeStruct((1, NPAD), jnp.float32),
            jax.ShapeDtypeStruct((1, NPAD), jnp.float32),
            jax.ShapeDtypeStruct((1, 4 * NPAD), jnp.float32),
        ],
    )(den_p, deg_p, c_p)


def _k5a_body(ex_hbm, src_hbm, dst_hbm, den_hbm, deg_hbm,
              al_hbm, a1_hbm, a3_hbm, asum_hbm,
              exv, srcv, dstv, denv, degv, a1v, a3v, asumv):
    wid = lax.axis_index("s") * 2 + lax.axis_index("c")
    base = pl.multiple_of(wid * EW, 8)
    pltpu.sync_copy(ex_hbm.at[pl.ds(base, EW)], exv)
    pltpu.sync_copy(src_hbm.at[pl.ds(base, EW)], srcv)
    pltpu.sync_copy(dst_hbm.at[pl.ds(base, EW)], dstv)
    pltpu.sync_copy(den_hbm, denv)
    pltpu.sync_copy(deg_hbm, degv)

    def zero(i, ref):
        ref[pl.ds(i * 16, 16)] = jnp.zeros((16,), jnp.float32)

    lax.fori_loop(0, 4 * NPAD // 16, lambda i, _: (zero(i, a1v), _)[1], None)
    lax.fori_loop(0, 4 * NPAD // 16, lambda i, _: (zero(i, a3v), _)[1], None)
    for b in range(8):
        asumv[pl.ds(b * 16, 16)] = jnp.zeros((16,), jnp.float32)

    lane = jnp.arange(16, dtype=jnp.int32)
    zf = jnp.zeros((16,), jnp.float32)

    def body(i, _):
        sl = pl.ds(i * 16, 16)
        d16 = dstv[sl]
        s16 = srcv[sl]
        den16 = plsc.load_gather(denv, [d16])
        a16 = exv[sl] / den16
        exv[sl] = a16
        deg16 = plsc.load_gather(degv, [d16])
        m16 = d16 // NB
        fl = m16 * NPAD + s16
        plsc.addupdate_scatter(a1v, [fl], a16)
        plsc.addupdate_scatter(a3v, [fl], deg16 * a16)
        p16 = (base + i * 16 + lane) // EB
        for b in range(B):
            plsc.addupdate(asumv.at[pl.ds(b * 16, 16)],
                           jnp.where(m16 == b, a16, zf))
            plsc.addupdate(asumv.at[pl.ds((4 + b) * 16, 16)],
                           jnp.where(p16 == b, a16, zf))
        return _

    lax.fori_loop(0, EW // 16, body, None)

    pltpu.sync_copy(exv, al_hbm.at[pl.ds(base, EW)])
    pltpu.sync_copy(a1v, a1_hbm.at[wid])
    pltpu.sync_copy(a3v, a3_hbm.at[wid])
    pltpu.sync_copy(asumv, asum_hbm.at[wid])


def _k5a(ex, src, dst, den1, deg1):
    mesh = plsc.VectorSubcoreMesh(core_axis_name="c", subcore_axis_name="s")
    f = pl.kernel(
        _k5a_body,
        out_type=(
            jax.ShapeDtypeStruct((EPAD,), jnp.float32),
            jax.ShapeDtypeStruct((NW, 4 * NPAD), jnp.float32),
            jax.ShapeDtypeStruct((NW, 4 * NPAD), jnp.float32),
            jax.ShapeDtypeStruct((NW, 128), jnp.float32),
        ),
        mesh=mesh,
        compiler_params=pltpu.CompilerParams(needs_layout_passes=False),
        scratch_types=[
            pltpu.VMEM((EW,), jnp.float32),
            pltpu.VMEM((EW,), jnp.int32),
            pltpu.VMEM((EW,), jnp.int32),
            pltpu.VMEM((NPAD,), jnp.float32),
            pltpu.VMEM((NPAD,), jnp.float32),
            pltpu.VMEM((4 * NPAD,), jnp.float32),
            pltpu.VMEM((4 * NPAD,), jnp.float32),
            pltpu.VMEM((128,), jnp.float32),
        ],
    )
    return f(ex, src, dst, den1, deg1)


def _k5b_body(al_hbm, src_hbm, dst_hbm, c_hbm, a2_hbm,
              alv, srcv, dstv, cv, a2v):
    wid = lax.axis_index("s") * 2 + lax.axis_index("c")
    base = pl.multiple_of(wid * EW, 8)
    pltpu.sync_copy(al_hbm.at[pl.ds(base, EW)], alv)
    pltpu.sync_copy(src_hbm.at[pl.ds(base, EW)], srcv)
    pltpu.sync_copy(dst_hbm.at[pl.ds(base, EW)], dstv)
    pltpu.sync_copy(c_hbm, cv)

    def zero(i, ref):
        ref[pl.ds(i * 16, 16)] = jnp.zeros((16,), jnp.float32)

    lax.fori_loop(0, 4 * NPAD // 16, lambda i, _: (zero(i, a2v), _)[1], None)

    def body(i, _):
        sl = pl.ds(i * 16, 16)
        a16 = alv[sl]
        d16 = dstv[sl]
        s16 = srcv[sl]
        for b in range(B):
            cb = plsc.load_gather(cv, [b * NPAD + d16])
            plsc.addupdate_scatter(a2v, [b * NPAD + s16], cb * a16)
        return _

    lax.fori_loop(0, EW // 16, body, None)
    pltpu.sync_copy(a2v, a2_hbm.at[wid])


def _k5b(al, src, dst, c1):
    mesh = plsc.VectorSubcoreMesh(core_axis_name="c", subcore_axis_name="s")
    f = pl.kernel(
        _k5b_body,
        out_type=jax.ShapeDtypeStruct((NW, 4 * NPAD), jnp.float32),
        mesh=mesh,
        compiler_params=pltpu.CompilerParams(needs_layout_passes=False),
        scratch_types=[
            pltpu.VMEM((EW,), jnp.float32),
            pltpu.VMEM((EW,), jnp.int32),
            pltpu.VMEM((EW,), jnp.int32),
            pltpu.VMEM((4 * NPAD,), jnp.float32),
            pltpu.VMEM((4 * NPAD,), jnp.float32),
        ],
    )
    return f(al, src, dst, c1)


def _k6_body(a1_ref, a2_ref, a3_ref, xl_ref, ehd_ref, ehp_ref, asum_ref,
             y2_ref, wgp_ref, bgp_ref, bgat_ref, wmsg_ref, bmsg_ref,
             wmsgi_ref, bmsgi_ref, wge_ref, bge_ref, wf1_ref, bf1_ref,
             wf2_ref, bf2_ref, ss_ref, out_ref):
    j = pl.program_id(0)
    a1 = jnp.sum(a1_ref[...], axis=0)
    a2 = jnp.sum(a2_ref[...], axis=0)
    a3 = jnp.sum(a3_ref[...], axis=0)
    astack = jnp.concatenate([a1, a2, a3], axis=0)
    contrib = jnp.dot(astack, xl_ref[...], preferred_element_type=jnp.float32)
    ss_ref[...] = jnp.where(j == 0, 0.0, ss_ref[...]) + contrib

    @pl.when(j == pl.num_programs(0) - 1)
    def _():
        ss = ss_ref[...]
        s1, s2, s3 = ss[0:4], ss[4:8], ss[8:12]
        asum = jnp.sum(jnp.sum(asum_ref[...], axis=0), axis=1, keepdims=True)
        ad = asum[0:4]
        ap = asum[4:8]
        ehd = jnp.sum(ehd_ref[...], axis=0)
        cnt = ehd[:, 511:512]
        bgat = bgat_ref[...]
        ehad = jnp.concatenate([ehd[:, 0:511], ad], axis=1)
        numer = (s1 + NB * bgat
                 + jnp.dot(s2 + cnt * bgat, wmsg_ref[...],
                           preferred_element_type=jnp.float32)
                 + jnp.dot(s3 + cnt * bgat, wmsgi_ref[...],
                           preferred_element_type=jnp.float32)
                 + jnp.dot(ehad, wge_ref[...],
                           preferred_element_type=jnp.float32)
                 + cnt * (bmsg_ref[...] + bmsgi_ref[...] + bge_ref[...]))
        out_nodes = numer / NB
        ehp = jnp.sum(ehp_ref[...], axis=0)
        out_edges = jnp.concatenate([ehp[:, 0:511], ap], axis=1) / EB
        gg = _lrelu(jnp.dot(y2_ref[...], wgp_ref[...],
                            preferred_element_type=jnp.float32) + bgp_ref[...], 0.01)
        pooled = jnp.concatenate([out_nodes, out_edges, gg], axis=1)
        o = _lrelu(jnp.dot(pooled, wf1_ref[...],
                           preferred_element_type=jnp.float32) + bf1_ref[...], 0.01)
        out_ref[...] = jax.nn.sigmoid(
            jnp.dot(o, wf2_ref[...], preferred_element_type=jnp.float32)
            + bf2_ref[...])


def _k6(a1_p, a2_p, a3_p, x_l, ehd, ehp, asum_p, y2, W_gp,
        b_gp, b_gat, W_msg, b_msg, W_msg_i, b_msg_i, W_ge, b_ge, W_f1,
        b_f1, W_f2, b_f2):
    BN = 2048
    nj = NPAD // BN
    full = lambda shape: pl.BlockSpec(shape, lambda j: tuple(0 for _ in shape))
    outs = pl.pallas_call(
        _k6_body,
        grid=(nj,),
        in_specs=[
            pl.BlockSpec((NW, B, BN), lambda j: (0, 0, j)),
            pl.BlockSpec((NW, B, BN), lambda j: (0, 0, j)),
            pl.BlockSpec((NW, B, BN), lambda j: (0, 0, j)),
            pl.BlockSpec((BN, 1024), lambda j: (j, 0)),
            full((4, B, 512)), full((4, B, 512)),
            full((NW, 8, 16)),
            full((B, 5)), full((5, 512)), full((1, 512)),
            full((1, 1024)), full((1024, 1024)), full((1, 1024)),
            full((1024, 1024)), full((1, 1024)),
            full((512, 1024)), full((1, 1024)),
            full((2048, 256)), full((1, 256)), full((256, 1)), full((1, 1)),
        ],
        out_specs=[
            full((12, 1024)),
            full((B, 1)),
        ],
        out_shape=[
            jax.ShapeDtypeStruct((12, 1024), jnp.float32),
            jax.ShapeDtypeStruct((B, 1), jnp.float32),
        ],
        compiler_params=pltpu.CompilerParams(
            dimension_semantics=("arbitrary",)),
    )(a1_p, a2_p, a3_p, x_l, ehd, ehp, asum_p, y2, W_gp, b_gp,
      b_gat, W_msg, b_msg, W_msg_i, b_msg_i, W_ge, b_ge, W_f1, b_f1,
      W_f2, b_f2)
    return outs[1]


def kernel(x, edge_index, edge_attr, y, batch, W_node, b_node, W_ep, b_ep,
           W_gp, b_gp, W_l, b_l, W_r, b_r, W_e, att, b_gat, W_msg, b_msg,
           W_msg_i, b_msg_i, W_ge, b_ge, W_f1, b_f1, W_f2, b_f2):
    f32 = jnp.float32
    loop = jnp.arange(N, dtype=edge_index.dtype)
    pad1 = jnp.zeros((EPAD - ETOT,), jnp.int32)
    src = jnp.concatenate([edge_index[0], loop, pad1])
    dst = jnp.concatenate([edge_index[1], loop, pad1])
    ea_pad = jnp.concatenate(
        [edge_attr, jnp.ones((N, 5), f32), jnp.zeros((EPAD - ETOT, 5), f32)], axis=0)
    ec = jnp.minimum(jnp.arange(EPAD, dtype=jnp.int32) // EB, B - 1)
    dst3 = dst.reshape(G, 1, BLK)
    ec3 = ec.reshape(G, 1, BLK)
    x2 = jnp.concatenate([x[:, None], jnp.zeros((NPAD - N, 1), f32)], axis=0)
    y2 = y.reshape(B, 5)

    r2 = lambda a: a.reshape(1, -1)

    bf = jnp.bfloat16
    x_l, xl_p, xr_p = _k1(x2, r2(W_node), r2(b_node), W_l.astype(bf),
                          r2(b_l), W_r.astype(bf), r2(b_r))
    we_bf = W_e.astype(jnp.bfloat16)
    part_blocks = [42, 42, 42, 41]
    lgs, ehds, ehps, mxs = [], [], [], []
    lo = 0
    for gb in part_blocks:
        hi = lo + gb * BLK
        gl_i, gr_i = _g1(xl_p, xr_p, src[lo:hi], dst[lo:hi])
        lg3, ehd_h, ehp_h, mxv_h = _k2(
            ea_pad[lo:hi], gl_i, gr_i,
            dst3[lo // BLK:hi // BLK], ec3[lo // BLK:hi // BLK],
            W_ep, r2(b_ep), we_bf, r2(att), lo)
        lgs.append(lg3.reshape(hi - lo))
        ehds.append(ehd_h)
        ehps.append(ehp_h)
        mxs.append(mxv_h[0, :16])
        lo = hi
    lg = jnp.concatenate(lgs)
    mx2 = jnp.stack(mxs)
    ex, den_p, deg_p, c_p = _k3(lg, src, dst, mx2)
    den1, deg1, c1 = _k4(den_p, deg_p, c_p)
    al, a1_p, a3_p, asum_p = _k5a(ex, src, dst, den1.reshape(NPAD),
                                  deg1.reshape(NPAD))
    a2_p = _k5b(al, src, dst, c1.reshape(4 * NPAD))
    out = _k6(a1_p.reshape(NW, B, NPAD), a2_p.reshape(NW, B, NPAD),
              a3_p.reshape(NW, B, NPAD), x_l, jnp.stack(ehds),
              jnp.stack(ehps), asum_p.reshape(NW, 8, 16), y2,
              W_gp, r2(b_gp), r2(b_gat), W_msg, r2(b_msg), W_msg_i,
              r2(b_msg_i), W_ge, r2(b_ge), W_f1, r2(b_f1), W_f2, r2(b_f2))
    return out
